# Initial kernel scaffold; baseline (speedup 1.0000x reference)
import jax
import jax.numpy as jnp
from jax.experimental import pallas as pl


def kernel(x, edge_index, r, params):
    raise NotImplementedError("write your pallas kernel here")


# speedup vs baseline: 1.4627x; 1.4627x over previous
import functools

import jax
import jax.numpy as jnp
from jax import lax
from jax.experimental import pallas as pl
from jax.experimental.pallas import tpu as pltpu
from jax.experimental.pallas import tpu_sc as plsc

_EPS = 1e-5


def _matmul(x, W, b, act=None, bm=1000, bn=None):
    M, K = x.shape
    Nout = W.shape[0]
    if bn is None:
        bn = Nout
    gm, gn = M // bm, Nout // bn
    b2 = b.reshape(1, Nout)

    def body(x_ref, w_ref, b_ref, o_ref):
        acc = lax.dot_general(x_ref[...], w_ref[...], (((1,), (1,)), ((), ())),
                              preferred_element_type=jnp.float32)
        acc = acc + b_ref[...]
        if act == "relu":
            acc = jnp.maximum(acc, 0.0)
        o_ref[...] = acc

    return pl.pallas_call(
        body,
        grid=(gn, gm),
        in_specs=[pl.BlockSpec((bm, K), lambda n, m: (m, 0)),
                  pl.BlockSpec((bn, K), lambda n, m: (n, 0)),
                  pl.BlockSpec((1, bn), lambda n, m: (0, n))],
        out_specs=pl.BlockSpec((bm, bn), lambda n, m: (m, n)),
        out_shape=jax.ShapeDtypeStruct((M, Nout), jnp.float32),
    )(x, W, b2)


def _matmul_kacc(x, W, b, bm=1000, bk=256):
    M, K = x.shape
    Nout = W.shape[0]
    gm = M // bm

    def body(x_ref, w_ref, b_ref, o_ref):
        acc = lax.dot_general(x_ref[...], w_ref[...], (((1,), (1,)), ((), ())),
                              preferred_element_type=jnp.float32)
        o_ref[...] = jnp.maximum(acc + b_ref[...], 0.0)

    return pl.pallas_call(
        body,
        grid=(gm,),
        in_specs=[pl.BlockSpec((bm, K), lambda m: (m, 0)),
                  pl.BlockSpec((Nout, K), lambda m: (0, 0)),
                  pl.BlockSpec((1, Nout), lambda m: (0, 0))],
        out_specs=pl.BlockSpec((bm, Nout), lambda m: (m, 0)),
        out_shape=jax.ShapeDtypeStruct((M, Nout), jnp.float32),
    )(x.astype(jnp.bfloat16), W.astype(jnp.bfloat16), b.reshape(1, Nout))


def _stats(p, bm=2000):
    R, C = p.shape
    g = R // bm

    def body(p_ref, s_ref, q_ref):
        i = pl.program_id(0)
        t = p_ref[...]
        ts = jnp.sum(t, axis=0, keepdims=True)
        tq = jnp.sum(t * t, axis=0, keepdims=True)

        @pl.when(i == 0)
        def _():
            s_ref[...] = ts
            q_ref[...] = tq

        @pl.when(i > 0)
        def _():
            s_ref[...] += ts
            q_ref[...] += tq

    return pl.pallas_call(
        body,
        grid=(g,),
        in_specs=[pl.BlockSpec((bm, C), lambda i: (i, 0))],
        out_specs=[pl.BlockSpec((1, C), lambda i: (0, 0)),
                   pl.BlockSpec((1, C), lambda i: (0, 0))],
        out_shape=[jax.ShapeDtypeStruct((1, C), jnp.float32),
                   jax.ShapeDtypeStruct((1, C), jnp.float32)],
    )(p)


def _bn_coeffs(s_ref, q_ref, g_ref, b_ref, rows):
    mean = s_ref[...] / rows
    var = q_ref[...] / rows - mean * mean
    a = g_ref[...] / jnp.sqrt(var + _EPS)
    c = b_ref[...] - a * mean
    return a, c


def _softplus(x):
    return jnp.maximum(x, 0.0) + jnp.log1p(jnp.exp(-jnp.abs(x)))


def _sigmoid(x):
    return 1.0 / (1.0 + jnp.exp(-x))


def _bn_relu(p, s, q, g, b, rows, bm=2000):
    R, C = p.shape

    def body(p_ref, s_ref, q_ref, g_ref, b_ref, o_ref):
        a, c = _bn_coeffs(s_ref, q_ref, g_ref, b_ref, rows)
        o_ref[...] = jnp.maximum(a * p_ref[...] + c, 0.0)

    cmap = lambda i: (0, 0)
    return pl.pallas_call(
        body,
        grid=(R // bm,),
        in_specs=[pl.BlockSpec((bm, C), lambda i: (i, 0))] +
                 [pl.BlockSpec((1, C), cmap)] * 4,
        out_specs=pl.BlockSpec((bm, C), lambda i: (i, 0)),
        out_shape=jax.ShapeDtypeStruct((R, C), jnp.float32),
    )(p, s, q, g.reshape(1, C), b.reshape(1, C))


def _rbf(r, EIN=80, bm=2000):
    E = r.shape[0]
    step = 8.0 / (EIN - 1)
    gamma = 1.0 / step

    def body(r_ref, o_ref):
        rt = r_ref[...]
        d = jnp.sqrt(jnp.sum(rt * rt, axis=1, keepdims=True))
        centers = lax.broadcasted_iota(jnp.int32, (1, EIN), 1).astype(jnp.float32) * step
        diff = d - centers
        o_ref[...] = jnp.exp(-gamma * diff * diff)

    return pl.pallas_call(
        body,
        grid=(E // bm,),
        in_specs=[pl.BlockSpec((bm, 3), lambda i: (i, 0))],
        out_specs=pl.BlockSpec((bm, EIN), lambda i: (i, 0)),
        out_shape=jax.ShapeDtypeStruct((E, EIN), jnp.float32),
    )(r)


def _ye_stats(GA, GB, y, We, be, bm=2000):
    E, D = GA.shape
    K = y.shape[1]

    def body(ga_ref, gb_ref, y_ref, w_ref, b_ref, z_ref, s_ref, q_ref):
        i = pl.program_id(0)
        zt = lax.dot_general(y_ref[...], w_ref[...], (((1,), (1,)), ((), ())),
                             preferred_element_type=jnp.float32)
        zt = zt + ga_ref[...] + gb_ref[...] + b_ref[...]
        z_ref[...] = zt
        ts = jnp.sum(zt, axis=0, keepdims=True)
        tq = jnp.sum(zt * zt, axis=0, keepdims=True)

        @pl.when(i == 0)
        def _():
            s_ref[...] = ts
            q_ref[...] = tq

        @pl.when(i > 0)
        def _():
            s_ref[...] += ts
            q_ref[...] += tq

    return pl.pallas_call(
        body,
        grid=(E // bm,),
        in_specs=[pl.BlockSpec((bm, D), lambda i: (i, 0)),
                  pl.BlockSpec((bm, D), lambda i: (i, 0)),
                  pl.BlockSpec((bm, K), lambda i: (i, 0)),
                  pl.BlockSpec((D, K), lambda i: (0, 0)),
                  pl.BlockSpec((1, D), lambda i: (0, 0))],
        out_specs=[pl.BlockSpec((bm, D), lambda i: (i, 0)),
                   pl.BlockSpec((1, D), lambda i: (0, 0)),
                   pl.BlockSpec((1, D), lambda i: (0, 0))],
        out_shape=[jax.ShapeDtypeStruct((E, D), jnp.float32),
                   jax.ShapeDtypeStruct((1, D), jnp.float32),
                   jax.ShapeDtypeStruct((1, D), jnp.float32)],
    )(GA, GB, y, We, be.reshape(1, D))


def _gate(z, s, q, gm, bmp, E, bm=2000):
    D = z.shape[1]
    H = D // 2
    Hh = H // 2

    def body(z_ref, s_ref, q_ref, g_ref, b_ref, m0_ref, m1_ref):
        a, c = _bn_coeffs(s_ref, q_ref, g_ref, b_ref, E)
        zn = a * z_ref[...] + c
        hf = zn[:, :H]
        hs = zn[:, H:]
        m = _sigmoid(hf) * _softplus(hs)
        m0_ref[...] = m[:, :Hh]
        m1_ref[...] = m[:, Hh:]

    cmap = lambda i: (0, 0)
    return pl.pallas_call(
        body,
        grid=(E // bm,),
        in_specs=[pl.BlockSpec((bm, D), lambda i: (i, 0))] +
                 [pl.BlockSpec((1, D), cmap)] * 4,
        out_specs=[pl.BlockSpec((bm, Hh), lambda i: (i, 0)),
                   pl.BlockSpec((bm, Hh), lambda i: (i, 0))],
        out_shape=[jax.ShapeDtypeStruct((E, Hh), jnp.float32),
                   jax.ShapeDtypeStruct((E, Hh), jnp.float32)],
    )(z, s, q, gm.reshape(1, D), bmp.reshape(1, D))


def _residual(h, a0, a1, s0, q0, s1, q1, g, b, rows, bm=2000):
    N, C = h.shape
    Hh = C // 2

    def body(h_ref, a0_ref, a1_ref, s0_ref, q0_ref, s1_ref, q1_ref,
             g_ref, b_ref, o_ref):
        g0 = g_ref[:, :Hh]
        g1 = g_ref[:, Hh:]
        b0 = b_ref[:, :Hh]
        b1 = b_ref[:, Hh:]
        ca0, cc0 = _bn_coeffs(s0_ref, q0_ref, g0, b0, rows)
        ca1, cc1 = _bn_coeffs(s1_ref, q1_ref, g1, b1, rows)
        n0 = ca0 * a0_ref[...] + cc0
        n1 = ca1 * a1_ref[...] + cc1
        aggn = jnp.concatenate([n0, n1], axis=1)
        o_ref[...] = _softplus(h_ref[...] + aggn)

    cmap = lambda i: (0, 0)
    return pl.pallas_call(
        body,
        grid=(N // bm,),
        in_specs=[pl.BlockSpec((bm, C), lambda i: (i, 0)),
                  pl.BlockSpec((bm, Hh), lambda i: (i, 0)),
                  pl.BlockSpec((bm, Hh), lambda i: (i, 0)),
                  pl.BlockSpec((1, Hh), cmap), pl.BlockSpec((1, Hh), cmap),
                  pl.BlockSpec((1, Hh), cmap), pl.BlockSpec((1, Hh), cmap),
                  pl.BlockSpec((1, C), cmap), pl.BlockSpec((1, C), cmap)],
        out_specs=pl.BlockSpec((bm, C), lambda i: (i, 0)),
        out_shape=jax.ShapeDtypeStruct((N, C), jnp.float32),
    )(h, a0, a1, s0, q0, s1, q1, g.reshape(1, C), b.reshape(1, C))


def _sc_gather(A, B, src, dst):
    E = src.shape[0]
    D = A.shape[1]
    NW = 32
    per = E // NW
    CH = 40
    NCH = per // CH
    mesh = plsc.VectorSubcoreMesh(core_axis_name="c", subcore_axis_name="s")

    @functools.partial(
        pl.kernel, mesh=mesh,
        out_type=[jax.ShapeDtypeStruct((E, D), jnp.float32),
                  jax.ShapeDtypeStruct((E, D), jnp.float32)],
        scratch_types=[pltpu.VMEM((CH,), jnp.int32),
                       pltpu.VMEM((CH,), jnp.int32),
                       pltpu.VMEM((CH, D), jnp.float32),
                       pltpu.VMEM((CH, D), jnp.float32),
                       pltpu.SemaphoreType.DMA,
                       pltpu.SemaphoreType.DMA])
    def k(a_hbm, b_hbm, s_hbm, d_hbm, oa_hbm, ob_hbm, si, di, ba, bb,
          semA, semB):
        wid = lax.axis_index("s") * 2 + lax.axis_index("c")
        base = wid * per

        def chunk(t, carry):
            off = base + t * CH
            pltpu.sync_copy(s_hbm.at[pl.ds(off, CH)], si)
            pltpu.sync_copy(d_hbm.at[pl.ds(off, CH)], di)
            ca = pltpu.async_copy(a_hbm.at[si], ba, semA)
            cb = pltpu.async_copy(b_hbm.at[di], bb, semB)
            ca.wait()
            cb.wait()
            pltpu.sync_copy(ba, oa_hbm.at[pl.ds(off, CH)])
            pltpu.sync_copy(bb, ob_hbm.at[pl.ds(off, CH)])
            return carry

        lax.fori_loop(0, NCH, chunk, 0)

    return k(A, B, src, dst)


def _sc_scatter(m0, m1, dst, N):
    E, Hh = m0.shape
    NS = 16
    per = E // NS
    CH = 200
    NCH = per // CH
    RB = 200
    NRB = N // RB
    NRB_PER = -(-NRB // NS)
    mesh = plsc.VectorSubcoreMesh(core_axis_name="c", subcore_axis_name="s")

    @functools.partial(
        pl.kernel, mesh=mesh,
        out_type=[jax.ShapeDtypeStruct((N, Hh), jnp.float32),
                  jax.ShapeDtypeStruct((N, Hh), jnp.float32)],
        scratch_types=[pltpu.VMEM((CH,), jnp.int32),
                       pltpu.VMEM((CH, Hh), jnp.float32),
                       pltpu.VMEM_SHARED((N, Hh), jnp.float32)])
    def k(m0_hbm, m1_hbm, d_hbm, o0_hbm, o1_hbm, idxv, mbuf, acc):
        cid = lax.axis_index("c")
        sid = lax.axis_index("s")

        def zrow(e, carry):
            for j in range(Hh // 16):
                mbuf[e, pl.ds(j * 16, 16)] = jnp.zeros((16,), jnp.float32)
            return carry

        lax.fori_loop(0, CH, zrow, 0)

        def zcp(t, carry):
            c = sid + t * NS

            @pl.when(c < NRB)
            def _():
                pltpu.sync_copy(mbuf.at[pl.ds(0, RB)],
                                acc.at[pl.ds(c * RB, RB)])

            return carry

        lax.fori_loop(0, NRB_PER, zcp, 0)
        plsc.subcore_barrier()

        def run_half(m_hbm, o_hbm):
            def chunk(t, carry):
                off = sid * per + t * CH
                pltpu.sync_copy(d_hbm.at[pl.ds(off, CH)], idxv)
                pltpu.sync_copy(m_hbm.at[pl.ds(off, CH)], mbuf)
                pltpu.sync_copy(mbuf, acc.at[idxv], add=True)
                return carry

            lax.fori_loop(0, NCH, chunk, 0)
            plsc.subcore_barrier()

            def wb(t, carry):
                c = sid + t * NS

                @pl.when(c < NRB)
                def _():
                    row = c * RB
                    pltpu.sync_copy(acc.at[pl.ds(row, RB)],
                                    mbuf.at[pl.ds(0, RB)])
                    pltpu.sync_copy(mbuf.at[pl.ds(0, RB)],
                                    o_hbm.at[pl.ds(row, RB)])

                return carry

            lax.fori_loop(0, NRB_PER, wb, 0)

        @pl.when(cid == 0)
        def _():
            run_half(m0_hbm, o0_hbm)

        @pl.when(cid == 1)
        def _():
            run_half(m1_hbm, o1_hbm)

    return k(m0, m1, dst)


def kernel(x, edge_index, r, params):
    N, AIN = x.shape
    E = r.shape[0]
    src = edge_index[0]
    dst = edge_index[1]

    Wa, ba, ga, bba = params["atom"]
    p = _matmul(x, Wa, ba)
    s, q = _stats(p)
    h = _bn_relu(p, s, q, ga, bba, N)

    rb = _rbf(r)
    W1, b1, g1, be1 = params["e1"]
    p1 = _matmul(rb, W1, b1, bm=2000)
    s, q = _stats(p1)
    y = _bn_relu(p1, s, q, g1, be1, E)
    W2, b2, g2, be2 = params["e2"]
    p2 = _matmul(y, W2, b2, bm=2000)
    s, q = _stats(p2)
    y = _bn_relu(p2, s, q, g2, be2, E)

    for cp in params["convs"]:
        A = _matmul(h, cp["Wsrc"], cp["bsrc"])
        B = _matmul(h, cp["Wdst"], cp["bdst"])
        GA, GB = _sc_gather(A, B, src, dst)
        z, s, q = _ye_stats(GA, GB, y, cp["We"], cp["be"])
        m0, m1 = _gate(z, s, q, cp["gm"], cp["bm"], E)
        a0, a1 = _sc_scatter(m0, m1, dst, N)
        s0, q0 = _stats(a0, bm=2000)
        s1, q1 = _stats(a1, bm=2000)
        h = _residual(h, a0, a1, s0, q0, s1, q1, cp["g2"], cp["b2"], N)

    W1f, b1f = params["fc1"]
    W2f, b2f = params["fc2"]
    W3f, b3f = params["fc3"]
    h = _matmul(h, W1f, b1f, act="relu")
    h = _matmul(h, W2f, b2f, act="relu")
    h = _matmul_kacc(h, W3f, b3f, bm=200)
    return h.reshape(-1, 100, 100)


# speedup vs baseline: 1.8696x; 1.2782x over previous
import functools

import jax
import jax.numpy as jnp
from jax import lax
from jax.experimental import pallas as pl
from jax.experimental.pallas import tpu as pltpu
from jax.experimental.pallas import tpu_sc as plsc

_EPS = 1e-5


def _matmul(x, W, b, act=None, bm=1000, bn=None):
    M, K = x.shape
    Nout = W.shape[0]
    if bn is None:
        bn = Nout
    gm, gn = M // bm, Nout // bn
    b2 = b.reshape(1, Nout)

    def body(x_ref, w_ref, b_ref, o_ref):
        acc = lax.dot_general(x_ref[...], w_ref[...], (((1,), (1,)), ((), ())),
                              preferred_element_type=jnp.float32)
        acc = acc + b_ref[...]
        if act == "relu":
            acc = jnp.maximum(acc, 0.0)
        o_ref[...] = acc

    return pl.pallas_call(
        body,
        grid=(gn, gm),
        in_specs=[pl.BlockSpec((bm, K), lambda n, m: (m, 0)),
                  pl.BlockSpec((bn, K), lambda n, m: (n, 0)),
                  pl.BlockSpec((1, bn), lambda n, m: (0, n))],
        out_specs=pl.BlockSpec((bm, bn), lambda n, m: (m, n)),
        out_shape=jax.ShapeDtypeStruct((M, Nout), jnp.float32),
    )(x, W, b2)


def _matmul_kacc(x, W, b, bm=1000, bk=256):
    M, K = x.shape
    Nout = W.shape[0]
    gm = M // bm

    def body(x_ref, w_ref, b_ref, o_ref):
        acc = lax.dot_general(x_ref[...], w_ref[...], (((1,), (1,)), ((), ())),
                              preferred_element_type=jnp.float32)
        o_ref[...] = jnp.maximum(acc + b_ref[...], 0.0)

    return pl.pallas_call(
        body,
        grid=(gm,),
        in_specs=[pl.BlockSpec((bm, K), lambda m: (m, 0)),
                  pl.BlockSpec((Nout, K), lambda m: (0, 0)),
                  pl.BlockSpec((1, Nout), lambda m: (0, 0))],
        out_specs=pl.BlockSpec((bm, Nout), lambda m: (m, 0)),
        out_shape=jax.ShapeDtypeStruct((M, Nout), jnp.float32),
    )(x.astype(jnp.bfloat16), W.astype(jnp.bfloat16), b.reshape(1, Nout))


def _stats(p, bm=2000):
    R, C = p.shape
    g = R // bm

    def body(p_ref, s_ref, q_ref):
        i = pl.program_id(0)
        t = p_ref[...]
        ts = jnp.sum(t, axis=0, keepdims=True)
        tq = jnp.sum(t * t, axis=0, keepdims=True)

        @pl.when(i == 0)
        def _():
            s_ref[...] = ts
            q_ref[...] = tq

        @pl.when(i > 0)
        def _():
            s_ref[...] += ts
            q_ref[...] += tq

    return pl.pallas_call(
        body,
        grid=(g,),
        in_specs=[pl.BlockSpec((bm, C), lambda i: (i, 0))],
        out_specs=[pl.BlockSpec((1, C), lambda i: (0, 0)),
                   pl.BlockSpec((1, C), lambda i: (0, 0))],
        out_shape=[jax.ShapeDtypeStruct((1, C), jnp.float32),
                   jax.ShapeDtypeStruct((1, C), jnp.float32)],
    )(p)


def _bn_coeffs(s_ref, q_ref, g_ref, b_ref, rows):
    mean = s_ref[...] / rows
    var = q_ref[...] / rows - mean * mean
    a = g_ref[...] / jnp.sqrt(var + _EPS)
    c = b_ref[...] - a * mean
    return a, c


def _softplus(x):
    return jnp.maximum(x, 0.0) + jnp.log1p(jnp.exp(-jnp.abs(x)))


def _sigmoid(x):
    return 1.0 / (1.0 + jnp.exp(-x))


def _bn_relu(p, s, q, g, b, rows, bm=2000):
    R, C = p.shape

    def body(p_ref, s_ref, q_ref, g_ref, b_ref, o_ref):
        a, c = _bn_coeffs(s_ref, q_ref, g_ref, b_ref, rows)
        o_ref[...] = jnp.maximum(a * p_ref[...] + c, 0.0)

    cmap = lambda i: (0, 0)
    return pl.pallas_call(
        body,
        grid=(R // bm,),
        in_specs=[pl.BlockSpec((bm, C), lambda i: (i, 0))] +
                 [pl.BlockSpec((1, C), cmap)] * 4,
        out_specs=pl.BlockSpec((bm, C), lambda i: (i, 0)),
        out_shape=jax.ShapeDtypeStruct((R, C), jnp.float32),
    )(p, s, q, g.reshape(1, C), b.reshape(1, C))


def _rbf(r, EIN=80, bm=2000):
    E = r.shape[0]
    step = 8.0 / (EIN - 1)
    gamma = 1.0 / step

    def body(r_ref, o_ref):
        rt = r_ref[...]
        d = jnp.sqrt(jnp.sum(rt * rt, axis=1, keepdims=True))
        centers = lax.broadcasted_iota(jnp.int32, (1, EIN), 1).astype(jnp.float32) * step
        diff = d - centers
        o_ref[...] = jnp.exp(-gamma * diff * diff)

    return pl.pallas_call(
        body,
        grid=(E // bm,),
        in_specs=[pl.BlockSpec((bm, 3), lambda i: (i, 0))],
        out_specs=pl.BlockSpec((bm, EIN), lambda i: (i, 0)),
        out_shape=jax.ShapeDtypeStruct((E, EIN), jnp.float32),
    )(r)


def _ye_stats(Hs, Hd, y, Wsrc, Wdst, We, bsum, bm=2000):
    E, K = Hs.shape
    D = We.shape[0]
    dn = (((1,), (1,)), ((), ()))

    def body(hs_ref, hd_ref, y_ref, ws_ref, wd_ref, we_ref, b_ref,
             z_ref, s_ref, q_ref):
        i = pl.program_id(0)
        zt = lax.dot_general(hs_ref[...], ws_ref[...], dn,
                             preferred_element_type=jnp.float32)
        zt += lax.dot_general(hd_ref[...], wd_ref[...], dn,
                              preferred_element_type=jnp.float32)
        zt += lax.dot_general(y_ref[...], we_ref[...], dn,
                              preferred_element_type=jnp.float32)
        zt += b_ref[...]
        z_ref[...] = zt
        ts = jnp.sum(zt, axis=0, keepdims=True)
        tq = jnp.sum(zt * zt, axis=0, keepdims=True)

        @pl.when(i == 0)
        def _():
            s_ref[...] = ts
            q_ref[...] = tq

        @pl.when(i > 0)
        def _():
            s_ref[...] += ts
            q_ref[...] += tq

    return pl.pallas_call(
        body,
        grid=(E // bm,),
        in_specs=[pl.BlockSpec((bm, K), lambda i: (i, 0)),
                  pl.BlockSpec((bm, K), lambda i: (i, 0)),
                  pl.BlockSpec((bm, K), lambda i: (i, 0)),
                  pl.BlockSpec((D, K), lambda i: (0, 0)),
                  pl.BlockSpec((D, K), lambda i: (0, 0)),
                  pl.BlockSpec((D, K), lambda i: (0, 0)),
                  pl.BlockSpec((1, D), lambda i: (0, 0))],
        out_specs=[pl.BlockSpec((bm, D), lambda i: (i, 0)),
                   pl.BlockSpec((1, D), lambda i: (0, 0)),
                   pl.BlockSpec((1, D), lambda i: (0, 0))],
        out_shape=[jax.ShapeDtypeStruct((E, D), jnp.float32),
                   jax.ShapeDtypeStruct((1, D), jnp.float32),
                   jax.ShapeDtypeStruct((1, D), jnp.float32)],
    )(Hs, Hd, y, Wsrc, Wdst, We, bsum.reshape(1, D))


def _gate(z, s, q, gm, bmp, E, bm=2000):
    D = z.shape[1]
    H = D // 2
    Hh = H // 2

    def body(z_ref, s_ref, q_ref, g_ref, b_ref, m0_ref, m1_ref):
        a, c = _bn_coeffs(s_ref, q_ref, g_ref, b_ref, E)
        zn = a * z_ref[...] + c
        hf = zn[:, :H]
        hs = zn[:, H:]
        m = _sigmoid(hf) * _softplus(hs)
        m0_ref[...] = m[:, :Hh]
        m1_ref[...] = m[:, Hh:]

    cmap = lambda i: (0, 0)
    return pl.pallas_call(
        body,
        grid=(E // bm,),
        in_specs=[pl.BlockSpec((bm, D), lambda i: (i, 0))] +
                 [pl.BlockSpec((1, D), cmap)] * 4,
        out_specs=[pl.BlockSpec((bm, Hh), lambda i: (i, 0)),
                   pl.BlockSpec((bm, Hh), lambda i: (i, 0))],
        out_shape=[jax.ShapeDtypeStruct((E, Hh), jnp.float32),
                   jax.ShapeDtypeStruct((E, Hh), jnp.float32)],
    )(z, s, q, gm.reshape(1, D), bmp.reshape(1, D))


def _residual(h, a0, a1, s0, q0, s1, q1, g, b, rows, bm=2000):
    N, C = h.shape
    Hh = C // 2

    def body(h_ref, a0_ref, a1_ref, s0_ref, q0_ref, s1_ref, q1_ref,
             g_ref, b_ref, o_ref):
        g0 = g_ref[:, :Hh]
        g1 = g_ref[:, Hh:]
        b0 = b_ref[:, :Hh]
        b1 = b_ref[:, Hh:]
        ca0, cc0 = _bn_coeffs(s0_ref, q0_ref, g0, b0, rows)
        ca1, cc1 = _bn_coeffs(s1_ref, q1_ref, g1, b1, rows)
        n0 = ca0 * a0_ref[...] + cc0
        n1 = ca1 * a1_ref[...] + cc1
        aggn = jnp.concatenate([n0, n1], axis=1)
        o_ref[...] = _softplus(h_ref[...] + aggn)

    cmap = lambda i: (0, 0)
    return pl.pallas_call(
        body,
        grid=(N // bm,),
        in_specs=[pl.BlockSpec((bm, C), lambda i: (i, 0)),
                  pl.BlockSpec((bm, Hh), lambda i: (i, 0)),
                  pl.BlockSpec((bm, Hh), lambda i: (i, 0)),
                  pl.BlockSpec((1, Hh), cmap), pl.BlockSpec((1, Hh), cmap),
                  pl.BlockSpec((1, Hh), cmap), pl.BlockSpec((1, Hh), cmap),
                  pl.BlockSpec((1, C), cmap), pl.BlockSpec((1, C), cmap)],
        out_specs=pl.BlockSpec((bm, C), lambda i: (i, 0)),
        out_shape=jax.ShapeDtypeStruct((N, C), jnp.float32),
    )(h, a0, a1, s0, q0, s1, q1, g.reshape(1, C), b.reshape(1, C))


def _sc_gather(A, B, src, dst):
    E = src.shape[0]
    D = A.shape[1]
    NW = 32
    per = E // NW
    CH = 200 if D <= 256 else 40
    NCH = per // CH
    mesh = plsc.VectorSubcoreMesh(core_axis_name="c", subcore_axis_name="s")

    @functools.partial(
        pl.kernel, mesh=mesh,
        out_type=[jax.ShapeDtypeStruct((E, D), jnp.float32),
                  jax.ShapeDtypeStruct((E, D), jnp.float32)],
        scratch_types=[pltpu.VMEM((CH,), jnp.int32),
                       pltpu.VMEM((CH,), jnp.int32),
                       pltpu.VMEM((CH, D), jnp.float32),
                       pltpu.VMEM((CH, D), jnp.float32),
                       pltpu.SemaphoreType.DMA,
                       pltpu.SemaphoreType.DMA])
    def k(a_hbm, b_hbm, s_hbm, d_hbm, oa_hbm, ob_hbm, si, di, ba, bb,
          semA, semB):
        wid = lax.axis_index("s") * 2 + lax.axis_index("c")
        base = wid * per

        def chunk(t, carry):
            off = base + t * CH
            pltpu.sync_copy(s_hbm.at[pl.ds(off, CH)], si)
            pltpu.sync_copy(d_hbm.at[pl.ds(off, CH)], di)
            ca = pltpu.async_copy(a_hbm.at[si], ba, semA)
            cb = pltpu.async_copy(b_hbm.at[di], bb, semB)
            ca.wait()
            cb.wait()
            pltpu.sync_copy(ba, oa_hbm.at[pl.ds(off, CH)])
            pltpu.sync_copy(bb, ob_hbm.at[pl.ds(off, CH)])
            return carry

        lax.fori_loop(0, NCH, chunk, 0)

    return k(A, B, src, dst)


def _sc_scatter(m0, m1, dst, N):
    E, Hh = m0.shape
    NS = 16
    per = E // NS
    CH = 200
    NCH = per // CH
    RB = 200
    NRB = N // RB
    NRB_PER = -(-NRB // NS)
    mesh = plsc.VectorSubcoreMesh(core_axis_name="c", subcore_axis_name="s")

    @functools.partial(
        pl.kernel, mesh=mesh,
        out_type=[jax.ShapeDtypeStruct((N, Hh), jnp.float32),
                  jax.ShapeDtypeStruct((N, Hh), jnp.float32)],
        scratch_types=[pltpu.VMEM((CH,), jnp.int32),
                       pltpu.VMEM((CH, Hh), jnp.float32),
                       pltpu.VMEM_SHARED((N, Hh), jnp.float32)])
    def k(m0_hbm, m1_hbm, d_hbm, o0_hbm, o1_hbm, idxv, mbuf, acc):
        cid = lax.axis_index("c")
        sid = lax.axis_index("s")

        def zrow(e, carry):
            for j in range(Hh // 16):
                mbuf[e, pl.ds(j * 16, 16)] = jnp.zeros((16,), jnp.float32)
            return carry

        lax.fori_loop(0, CH, zrow, 0)

        def zcp(t, carry):
            c = sid + t * NS

            @pl.when(c < NRB)
            def _():
                pltpu.sync_copy(mbuf.at[pl.ds(0, RB)],
                                acc.at[pl.ds(c * RB, RB)])

            return carry

        lax.fori_loop(0, NRB_PER, zcp, 0)
        plsc.subcore_barrier()

        def run_half(m_hbm, o_hbm):
            def chunk(t, carry):
                off = sid * per + t * CH
                pltpu.sync_copy(d_hbm.at[pl.ds(off, CH)], idxv)
                pltpu.sync_copy(m_hbm.at[pl.ds(off, CH)], mbuf)
                pltpu.sync_copy(mbuf, acc.at[idxv], add=True)
                return carry

            lax.fori_loop(0, NCH, chunk, 0)
            plsc.subcore_barrier()

            def wb(t, carry):
                c = sid + t * NS

                @pl.when(c < NRB)
                def _():
                    row = c * RB
                    pltpu.sync_copy(acc.at[pl.ds(row, RB)],
                                    mbuf.at[pl.ds(0, RB)])
                    pltpu.sync_copy(mbuf.at[pl.ds(0, RB)],
                                    o_hbm.at[pl.ds(row, RB)])

                return carry

            lax.fori_loop(0, NRB_PER, wb, 0)

        @pl.when(cid == 0)
        def _():
            run_half(m0_hbm, o0_hbm)

        @pl.when(cid == 1)
        def _():
            run_half(m1_hbm, o1_hbm)

    return k(m0, m1, dst)


def kernel(x, edge_index, r, params):
    N, AIN = x.shape
    E = r.shape[0]
    src = edge_index[0]
    dst = edge_index[1]

    Wa, ba, ga, bba = params["atom"]
    p = _matmul(x, Wa, ba)
    s, q = _stats(p)
    h = _bn_relu(p, s, q, ga, bba, N)

    rb = _rbf(r)
    W1, b1, g1, be1 = params["e1"]
    p1 = _matmul(rb, W1, b1, bm=2000)
    s, q = _stats(p1)
    y = _bn_relu(p1, s, q, g1, be1, E)
    W2, b2, g2, be2 = params["e2"]
    p2 = _matmul(y, W2, b2, bm=2000)
    s, q = _stats(p2)
    y = _bn_relu(p2, s, q, g2, be2, E)

    for cp in params["convs"]:
        Hs, Hd = _sc_gather(h, h, src, dst)
        bsum = cp["bsrc"] + cp["bdst"] + cp["be"]
        z, s, q = _ye_stats(Hs, Hd, y, cp["Wsrc"], cp["Wdst"], cp["We"], bsum)
        m0, m1 = _gate(z, s, q, cp["gm"], cp["bm"], E)
        a0, a1 = _sc_scatter(m0, m1, dst, N)
        s0, q0 = _stats(a0, bm=2000)
        s1, q1 = _stats(a1, bm=2000)
        h = _residual(h, a0, a1, s0, q0, s1, q1, cp["g2"], cp["b2"], N)

    W1f, b1f = params["fc1"]
    W2f, b2f = params["fc2"]
    W3f, b3f = params["fc3"]
    h = _matmul(h, W1f, b1f, act="relu")
    h = _matmul(h, W2f, b2f, act="relu")
    h = _matmul_kacc(h, W3f, b3f, bm=200)
    return h.reshape(-1, 100, 100)


# speedup vs baseline: 1.8800x; 1.0056x over previous
import functools

import jax
import jax.numpy as jnp
from jax import lax
from jax.experimental import pallas as pl
from jax.experimental.pallas import tpu as pltpu
from jax.experimental.pallas import tpu_sc as plsc

_EPS = 1e-5


def _matmul(x, W, b, act=None, bm=1000, bn=None):
    M, K = x.shape
    Nout = W.shape[0]
    if bn is None:
        bn = Nout
    gm, gn = M // bm, Nout // bn
    b2 = b.reshape(1, Nout)

    def body(x_ref, w_ref, b_ref, o_ref):
        acc = lax.dot_general(x_ref[...], w_ref[...], (((1,), (1,)), ((), ())),
                              preferred_element_type=jnp.float32)
        acc = acc + b_ref[...]
        if act == "relu":
            acc = jnp.maximum(acc, 0.0)
        o_ref[...] = acc

    return pl.pallas_call(
        body,
        grid=(gn, gm),
        in_specs=[pl.BlockSpec((bm, K), lambda n, m: (m, 0)),
                  pl.BlockSpec((bn, K), lambda n, m: (n, 0)),
                  pl.BlockSpec((1, bn), lambda n, m: (0, n))],
        out_specs=pl.BlockSpec((bm, bn), lambda n, m: (m, n)),
        out_shape=jax.ShapeDtypeStruct((M, Nout), jnp.float32),
    )(x, W, b2)


def _matmul_kacc(x, W, b, bm=1000, bk=256):
    M, K = x.shape
    Nout = W.shape[0]
    gm = M // bm

    def body(x_ref, w_ref, b_ref, o_ref):
        acc = lax.dot_general(x_ref[...], w_ref[...], (((1,), (1,)), ((), ())),
                              preferred_element_type=jnp.float32)
        acc = jnp.maximum(acc + b_ref[...], 0.0)
        o_ref[...] = acc.reshape(bm, 100, 100)

    return pl.pallas_call(
        body,
        grid=(gm,),
        in_specs=[pl.BlockSpec((bm, K), lambda m: (m, 0)),
                  pl.BlockSpec((Nout, K), lambda m: (0, 0)),
                  pl.BlockSpec((1, Nout), lambda m: (0, 0))],
        out_specs=pl.BlockSpec((bm, 100, 100), lambda m: (m, 0, 0),
                               pipeline_mode=pl.Buffered(buffer_count=1)),
        out_shape=jax.ShapeDtypeStruct((M, 100, 100), jnp.float32),
    )(x.astype(jnp.bfloat16), W.astype(jnp.bfloat16), b.reshape(1, Nout))


def _stats(p, bm=2000):
    R, C = p.shape
    g = R // bm

    def body(p_ref, s_ref, q_ref):
        i = pl.program_id(0)
        t = p_ref[...]
        ts = jnp.sum(t, axis=0, keepdims=True)
        tq = jnp.sum(t * t, axis=0, keepdims=True)

        @pl.when(i == 0)
        def _():
            s_ref[...] = ts
            q_ref[...] = tq

        @pl.when(i > 0)
        def _():
            s_ref[...] += ts
            q_ref[...] += tq

    return pl.pallas_call(
        body,
        grid=(g,),
        in_specs=[pl.BlockSpec((bm, C), lambda i: (i, 0))],
        out_specs=[pl.BlockSpec((1, C), lambda i: (0, 0)),
                   pl.BlockSpec((1, C), lambda i: (0, 0))],
        out_shape=[jax.ShapeDtypeStruct((1, C), jnp.float32),
                   jax.ShapeDtypeStruct((1, C), jnp.float32)],
    )(p)


def _bn_coeffs(s_ref, q_ref, g_ref, b_ref, rows):
    mean = s_ref[...] / rows
    var = q_ref[...] / rows - mean * mean
    a = g_ref[...] / jnp.sqrt(var + _EPS)
    c = b_ref[...] - a * mean
    return a, c


def _softplus(x):
    return jnp.maximum(x, 0.0) + jnp.log1p(jnp.exp(-jnp.abs(x)))


def _sigmoid(x):
    return 1.0 / (1.0 + jnp.exp(-x))


def _bn_relu(p, s, q, g, b, rows, bm=2000):
    R, C = p.shape

    def body(p_ref, s_ref, q_ref, g_ref, b_ref, o_ref):
        a, c = _bn_coeffs(s_ref, q_ref, g_ref, b_ref, rows)
        o_ref[...] = jnp.maximum(a * p_ref[...] + c, 0.0)

    cmap = lambda i: (0, 0)
    return pl.pallas_call(
        body,
        grid=(R // bm,),
        in_specs=[pl.BlockSpec((bm, C), lambda i: (i, 0))] +
                 [pl.BlockSpec((1, C), cmap)] * 4,
        out_specs=pl.BlockSpec((bm, C), lambda i: (i, 0)),
        out_shape=jax.ShapeDtypeStruct((R, C), jnp.float32),
    )(p, s, q, g.reshape(1, C), b.reshape(1, C))


def _rbf(r, EIN=80, bm=2000):
    E = r.shape[0]
    step = 8.0 / (EIN - 1)
    gamma = 1.0 / step

    def body(r_ref, o_ref):
        rt = r_ref[...]
        d = jnp.sqrt(jnp.sum(rt * rt, axis=1, keepdims=True))
        centers = lax.broadcasted_iota(jnp.int32, (1, EIN), 1).astype(jnp.float32) * step
        diff = d - centers
        o_ref[...] = jnp.exp(-gamma * diff * diff)

    return pl.pallas_call(
        body,
        grid=(E // bm,),
        in_specs=[pl.BlockSpec((bm, 3), lambda i: (i, 0))],
        out_specs=pl.BlockSpec((bm, EIN), lambda i: (i, 0)),
        out_shape=jax.ShapeDtypeStruct((E, EIN), jnp.float32),
    )(r)


def _ye_stats(Hs, Hd, y, Wsrc, Wdst, We, bsum, bm=2000):
    E, K = Hs.shape
    D = We.shape[0]
    dn = (((1,), (1,)), ((), ()))

    def body(hs_ref, hd_ref, y_ref, ws_ref, wd_ref, we_ref, b_ref,
             z_ref, s_ref, q_ref):
        i = pl.program_id(0)
        zt = lax.dot_general(hs_ref[...], ws_ref[...], dn,
                             preferred_element_type=jnp.float32)
        zt += lax.dot_general(hd_ref[...], wd_ref[...], dn,
                              preferred_element_type=jnp.float32)
        zt += lax.dot_general(y_ref[...], we_ref[...], dn,
                              preferred_element_type=jnp.float32)
        zt += b_ref[...]
        z_ref[...] = zt
        ts = jnp.sum(zt, axis=0, keepdims=True)
        tq = jnp.sum(zt * zt, axis=0, keepdims=True)

        @pl.when(i == 0)
        def _():
            s_ref[...] = ts
            q_ref[...] = tq

        @pl.when(i > 0)
        def _():
            s_ref[...] += ts
            q_ref[...] += tq

    return pl.pallas_call(
        body,
        grid=(E // bm,),
        in_specs=[pl.BlockSpec((bm, K), lambda i: (i, 0)),
                  pl.BlockSpec((bm, K), lambda i: (i, 0)),
                  pl.BlockSpec((bm, K), lambda i: (i, 0)),
                  pl.BlockSpec((D, K), lambda i: (0, 0)),
                  pl.BlockSpec((D, K), lambda i: (0, 0)),
                  pl.BlockSpec((D, K), lambda i: (0, 0)),
                  pl.BlockSpec((1, D), lambda i: (0, 0))],
        out_specs=[pl.BlockSpec((bm, D), lambda i: (i, 0)),
                   pl.BlockSpec((1, D), lambda i: (0, 0)),
                   pl.BlockSpec((1, D), lambda i: (0, 0))],
        out_shape=[jax.ShapeDtypeStruct((E, D), jnp.float32),
                   jax.ShapeDtypeStruct((1, D), jnp.float32),
                   jax.ShapeDtypeStruct((1, D), jnp.float32)],
    )(Hs, Hd, y, Wsrc, Wdst, We, bsum.reshape(1, D))


def _gate(z, s, q, gm, bmp, E, bm=2000):
    D = z.shape[1]
    H = D // 2
    Hh = H // 2

    def body(z_ref, s_ref, q_ref, g_ref, b_ref, m0_ref, m1_ref):
        a, c = _bn_coeffs(s_ref, q_ref, g_ref, b_ref, E)
        zn = a * z_ref[...] + c
        hf = zn[:, :H]
        hs = zn[:, H:]
        m = _sigmoid(hf) * _softplus(hs)
        m0_ref[...] = m[:, :Hh]
        m1_ref[...] = m[:, Hh:]

    cmap = lambda i: (0, 0)
    return pl.pallas_call(
        body,
        grid=(E // bm,),
        in_specs=[pl.BlockSpec((bm, D), lambda i: (i, 0))] +
                 [pl.BlockSpec((1, D), cmap)] * 4,
        out_specs=[pl.BlockSpec((bm, Hh), lambda i: (i, 0)),
                   pl.BlockSpec((bm, Hh), lambda i: (i, 0))],
        out_shape=[jax.ShapeDtypeStruct((E, Hh), jnp.float32),
                   jax.ShapeDtypeStruct((E, Hh), jnp.float32)],
    )(z, s, q, gm.reshape(1, D), bmp.reshape(1, D))


def _residual(h, a0, a1, s0, q0, s1, q1, g, b, rows, bm=2000):
    N, C = h.shape
    Hh = C // 2

    def body(h_ref, a0_ref, a1_ref, s0_ref, q0_ref, s1_ref, q1_ref,
             g_ref, b_ref, o_ref):
        g0 = g_ref[:, :Hh]
        g1 = g_ref[:, Hh:]
        b0 = b_ref[:, :Hh]
        b1 = b_ref[:, Hh:]
        ca0, cc0 = _bn_coeffs(s0_ref, q0_ref, g0, b0, rows)
        ca1, cc1 = _bn_coeffs(s1_ref, q1_ref, g1, b1, rows)
        n0 = ca0 * a0_ref[...] + cc0
        n1 = ca1 * a1_ref[...] + cc1
        aggn = jnp.concatenate([n0, n1], axis=1)
        o_ref[...] = _softplus(h_ref[...] + aggn)

    cmap = lambda i: (0, 0)
    return pl.pallas_call(
        body,
        grid=(N // bm,),
        in_specs=[pl.BlockSpec((bm, C), lambda i: (i, 0)),
                  pl.BlockSpec((bm, Hh), lambda i: (i, 0)),
                  pl.BlockSpec((bm, Hh), lambda i: (i, 0)),
                  pl.BlockSpec((1, Hh), cmap), pl.BlockSpec((1, Hh), cmap),
                  pl.BlockSpec((1, Hh), cmap), pl.BlockSpec((1, Hh), cmap),
                  pl.BlockSpec((1, C), cmap), pl.BlockSpec((1, C), cmap)],
        out_specs=pl.BlockSpec((bm, C), lambda i: (i, 0)),
        out_shape=jax.ShapeDtypeStruct((N, C), jnp.float32),
    )(h, a0, a1, s0, q0, s1, q1, g.reshape(1, C), b.reshape(1, C))


def _sc_gather(A, B, src, dst):
    E = src.shape[0]
    D = A.shape[1]
    NW = 32
    per = E // NW
    CH = 200 if D <= 256 else 40
    NCH = per // CH
    mesh = plsc.VectorSubcoreMesh(core_axis_name="c", subcore_axis_name="s")

    @functools.partial(
        pl.kernel, mesh=mesh,
        out_type=[jax.ShapeDtypeStruct((E, D), jnp.float32),
                  jax.ShapeDtypeStruct((E, D), jnp.float32)],
        scratch_types=[pltpu.VMEM((CH,), jnp.int32),
                       pltpu.VMEM((CH,), jnp.int32),
                       pltpu.VMEM((CH, D), jnp.float32),
                       pltpu.VMEM((CH, D), jnp.float32),
                       pltpu.SemaphoreType.DMA,
                       pltpu.SemaphoreType.DMA])
    def k(a_hbm, b_hbm, s_hbm, d_hbm, oa_hbm, ob_hbm, si, di, ba, bb,
          semA, semB):
        wid = lax.axis_index("s") * 2 + lax.axis_index("c")
        base = wid * per

        def chunk(t, carry):
            off = base + t * CH
            pltpu.sync_copy(s_hbm.at[pl.ds(off, CH)], si)
            pltpu.sync_copy(d_hbm.at[pl.ds(off, CH)], di)
            ca = pltpu.async_copy(a_hbm.at[si], ba, semA)
            cb = pltpu.async_copy(b_hbm.at[di], bb, semB)
            ca.wait()
            cb.wait()
            pltpu.sync_copy(ba, oa_hbm.at[pl.ds(off, CH)])
            pltpu.sync_copy(bb, ob_hbm.at[pl.ds(off, CH)])
            return carry

        lax.fori_loop(0, NCH, chunk, 0)

    return k(A, B, src, dst)


def _sc_scatter(m0, m1, dst, N):
    E, Hh = m0.shape
    NS = 16
    per = E // NS
    CH = 200
    NCH = per // CH
    RB = 200
    NRB = N // RB
    NRB_PER = -(-NRB // NS)
    mesh = plsc.VectorSubcoreMesh(core_axis_name="c", subcore_axis_name="s")

    @functools.partial(
        pl.kernel, mesh=mesh,
        out_type=[jax.ShapeDtypeStruct((N, Hh), jnp.float32),
                  jax.ShapeDtypeStruct((N, Hh), jnp.float32)],
        scratch_types=[pltpu.VMEM((CH,), jnp.int32),
                       pltpu.VMEM((CH, Hh), jnp.float32),
                       pltpu.VMEM_SHARED((N, Hh), jnp.float32)])
    def k(m0_hbm, m1_hbm, d_hbm, o0_hbm, o1_hbm, idxv, mbuf, acc):
        cid = lax.axis_index("c")
        sid = lax.axis_index("s")

        def zrow(e, carry):
            for j in range(Hh // 16):
                mbuf[e, pl.ds(j * 16, 16)] = jnp.zeros((16,), jnp.float32)
            return carry

        lax.fori_loop(0, CH, zrow, 0)

        def zcp(t, carry):
            c = sid + t * NS

            @pl.when(c < NRB)
            def _():
                pltpu.sync_copy(mbuf.at[pl.ds(0, RB)],
                                acc.at[pl.ds(c * RB, RB)])

            return carry

        lax.fori_loop(0, NRB_PER, zcp, 0)
        plsc.subcore_barrier()

        def run_half(m_hbm, o_hbm):
            def chunk(t, carry):
                off = sid * per + t * CH
                pltpu.sync_copy(d_hbm.at[pl.ds(off, CH)], idxv)
                pltpu.sync_copy(m_hbm.at[pl.ds(off, CH)], mbuf)
                pltpu.sync_copy(mbuf, acc.at[idxv], add=True)
                return carry

            lax.fori_loop(0, NCH, chunk, 0)
            plsc.subcore_barrier()

            def wb(t, carry):
                c = sid + t * NS

                @pl.when(c < NRB)
                def _():
                    row = c * RB
                    pltpu.sync_copy(acc.at[pl.ds(row, RB)],
                                    mbuf.at[pl.ds(0, RB)])
                    pltpu.sync_copy(mbuf.at[pl.ds(0, RB)],
                                    o_hbm.at[pl.ds(row, RB)])

                return carry

            lax.fori_loop(0, NRB_PER, wb, 0)

        @pl.when(cid == 0)
        def _():
            run_half(m0_hbm, o0_hbm)

        @pl.when(cid == 1)
        def _():
            run_half(m1_hbm, o1_hbm)

    return k(m0, m1, dst)


def kernel(x, edge_index, r, params):
    N, AIN = x.shape
    E = r.shape[0]
    src = edge_index[0]
    dst = edge_index[1]

    Wa, ba, ga, bba = params["atom"]
    p = _matmul(x, Wa, ba)
    s, q = _stats(p)
    h = _bn_relu(p, s, q, ga, bba, N)

    rb = _rbf(r)
    W1, b1, g1, be1 = params["e1"]
    p1 = _matmul(rb, W1, b1, bm=2000)
    s, q = _stats(p1)
    y = _bn_relu(p1, s, q, g1, be1, E)
    W2, b2, g2, be2 = params["e2"]
    p2 = _matmul(y, W2, b2, bm=2000)
    s, q = _stats(p2)
    y = _bn_relu(p2, s, q, g2, be2, E)

    for cp in params["convs"]:
        Hs, Hd = _sc_gather(h, h, src, dst)
        bsum = cp["bsrc"] + cp["bdst"] + cp["be"]
        z, s, q = _ye_stats(Hs, Hd, y, cp["Wsrc"], cp["Wdst"], cp["We"], bsum)
        m0, m1 = _gate(z, s, q, cp["gm"], cp["bm"], E)
        a0, a1 = _sc_scatter(m0, m1, dst, N)
        s0, q0 = _stats(a0, bm=2000)
        s1, q1 = _stats(a1, bm=2000)
        h = _residual(h, a0, a1, s0, q0, s1, q1, cp["g2"], cp["b2"], N)

    W1f, b1f = params["fc1"]
    W2f, b2f = params["fc2"]
    W3f, b3f = params["fc3"]
    h = _matmul(h, W1f, b1f, act="relu")
    h = _matmul(h, W2f, b2f, act="relu")
    return _matmul_kacc(h, W3f, b3f, bm=200)


# speedup vs baseline: 1.9826x; 1.0546x over previous
import functools

import jax
import jax.numpy as jnp
from jax import lax
from jax.experimental import pallas as pl
from jax.experimental.pallas import tpu as pltpu
from jax.experimental.pallas import tpu_sc as plsc

_EPS = 1e-5


def _matmul(x, W, b, act=None, bm=1000, bn=None):
    M, K = x.shape
    Nout = W.shape[0]
    if bn is None:
        bn = Nout
    gm, gn = M // bm, Nout // bn
    b2 = b.reshape(1, Nout)

    def body(x_ref, w_ref, b_ref, o_ref):
        acc = lax.dot_general(x_ref[...], w_ref[...], (((1,), (1,)), ((), ())),
                              preferred_element_type=jnp.float32)
        acc = acc + b_ref[...]
        if act == "relu":
            acc = jnp.maximum(acc, 0.0)
        o_ref[...] = acc

    return pl.pallas_call(
        body,
        grid=(gn, gm),
        in_specs=[pl.BlockSpec((bm, K), lambda n, m: (m, 0)),
                  pl.BlockSpec((bn, K), lambda n, m: (n, 0)),
                  pl.BlockSpec((1, bn), lambda n, m: (0, n))],
        out_specs=pl.BlockSpec((bm, bn), lambda n, m: (m, n)),
        out_shape=jax.ShapeDtypeStruct((M, Nout), jnp.float32),
    )(x, W, b2)


def _matmul_kacc(x, W, b, bm=1000, bk=256):
    M, K = x.shape
    Nout = W.shape[0]
    gm = M // bm

    def body(x_ref, w_ref, b_ref, o_ref):
        acc = lax.dot_general(x_ref[...], w_ref[...], (((1,), (1,)), ((), ())),
                              preferred_element_type=jnp.float32)
        acc = jnp.maximum(acc + b_ref[...], 0.0)
        o_ref[...] = acc.reshape(bm, 100, 100)

    return pl.pallas_call(
        body,
        grid=(gm,),
        in_specs=[pl.BlockSpec((bm, K), lambda m: (m, 0)),
                  pl.BlockSpec((Nout, K), lambda m: (0, 0)),
                  pl.BlockSpec((1, Nout), lambda m: (0, 0))],
        out_specs=pl.BlockSpec((bm, 100, 100), lambda m: (m, 0, 0),
                               pipeline_mode=pl.Buffered(buffer_count=1)),
        out_shape=jax.ShapeDtypeStruct((M, 100, 100), jnp.float32),
    )(x.astype(jnp.bfloat16), W.astype(jnp.bfloat16), b.reshape(1, Nout))


def _stats(p, bm=2000):
    R, C = p.shape
    g = R // bm

    def body(p_ref, s_ref, q_ref):
        i = pl.program_id(0)
        t = p_ref[...]
        ts = jnp.sum(t, axis=0, keepdims=True)
        tq = jnp.sum(t * t, axis=0, keepdims=True)

        @pl.when(i == 0)
        def _():
            s_ref[...] = ts
            q_ref[...] = tq

        @pl.when(i > 0)
        def _():
            s_ref[...] += ts
            q_ref[...] += tq

    return pl.pallas_call(
        body,
        grid=(g,),
        in_specs=[pl.BlockSpec((bm, C), lambda i: (i, 0))],
        out_specs=[pl.BlockSpec((1, C), lambda i: (0, 0)),
                   pl.BlockSpec((1, C), lambda i: (0, 0))],
        out_shape=[jax.ShapeDtypeStruct((1, C), jnp.float32),
                   jax.ShapeDtypeStruct((1, C), jnp.float32)],
    )(p)


def _bn_coeffs(s_ref, q_ref, g_ref, b_ref, rows):
    mean = s_ref[...] / rows
    var = q_ref[...] / rows - mean * mean
    a = g_ref[...] / jnp.sqrt(var + _EPS)
    c = b_ref[...] - a * mean
    return a, c


def _softplus(x):
    return jnp.maximum(x, 0.0) + jnp.log1p(jnp.exp(-jnp.abs(x)))


def _sigmoid(x):
    return 1.0 / (1.0 + jnp.exp(-x))


def _bn_relu(p, s, q, g, b, rows, bm=2000, out_dtype=jnp.float32):
    R, C = p.shape

    def body(p_ref, s_ref, q_ref, g_ref, b_ref, o_ref):
        a, c = _bn_coeffs(s_ref, q_ref, g_ref, b_ref, rows)
        o_ref[...] = jnp.maximum(a * p_ref[...] + c, 0.0).astype(out_dtype)

    cmap = lambda i: (0, 0)
    return pl.pallas_call(
        body,
        grid=(R // bm,),
        in_specs=[pl.BlockSpec((bm, C), lambda i: (i, 0))] +
                 [pl.BlockSpec((1, C), cmap)] * 4,
        out_specs=pl.BlockSpec((bm, C), lambda i: (i, 0)),
        out_shape=jax.ShapeDtypeStruct((R, C), out_dtype),
    )(p, s, q, g.reshape(1, C), b.reshape(1, C))


def _rbf(r, EIN=80, bm=2000):
    E = r.shape[0]
    step = 8.0 / (EIN - 1)
    gamma = 1.0 / step

    def body(r_ref, o_ref):
        rt = r_ref[...]
        d = jnp.sqrt(jnp.sum(rt * rt, axis=1, keepdims=True))
        centers = lax.broadcasted_iota(jnp.int32, (1, EIN), 1).astype(jnp.float32) * step
        diff = d - centers
        o_ref[...] = jnp.exp(-gamma * diff * diff)

    return pl.pallas_call(
        body,
        grid=(E // bm,),
        in_specs=[pl.BlockSpec((bm, 3), lambda i: (i, 0))],
        out_specs=pl.BlockSpec((bm, EIN), lambda i: (i, 0)),
        out_shape=jax.ShapeDtypeStruct((E, EIN), jnp.float32),
    )(r)


def _ye_stats(Hs, Hd, y, Wsrc, Wdst, We, bsum, bm=2000):
    E, K = Hs.shape
    D = We.shape[0]
    dn = (((1,), (1,)), ((), ()))

    def body(hs_ref, hd_ref, y_ref, ws_ref, wd_ref, we_ref, b_ref,
             z_ref, s_ref, q_ref):
        i = pl.program_id(0)
        zt = lax.dot_general(hs_ref[...], ws_ref[...], dn,
                             preferred_element_type=jnp.float32)
        zt += lax.dot_general(hd_ref[...], wd_ref[...], dn,
                              preferred_element_type=jnp.float32)
        zt += lax.dot_general(y_ref[...], we_ref[...].astype(jnp.bfloat16), dn,
                              preferred_element_type=jnp.float32)
        zt += b_ref[...]
        z_ref[...] = zt.astype(jnp.bfloat16)
        ts = jnp.sum(zt, axis=0, keepdims=True)
        tq = jnp.sum(zt * zt, axis=0, keepdims=True)

        @pl.when(i == 0)
        def _():
            s_ref[...] = ts
            q_ref[...] = tq

        @pl.when(i > 0)
        def _():
            s_ref[...] += ts
            q_ref[...] += tq

    return pl.pallas_call(
        body,
        grid=(E // bm,),
        in_specs=[pl.BlockSpec((bm, K), lambda i: (i, 0)),
                  pl.BlockSpec((bm, K), lambda i: (i, 0)),
                  pl.BlockSpec((bm, K), lambda i: (i, 0)),
                  pl.BlockSpec((D, K), lambda i: (0, 0)),
                  pl.BlockSpec((D, K), lambda i: (0, 0)),
                  pl.BlockSpec((D, K), lambda i: (0, 0)),
                  pl.BlockSpec((1, D), lambda i: (0, 0))],
        out_specs=[pl.BlockSpec((bm, D), lambda i: (i, 0)),
                   pl.BlockSpec((1, D), lambda i: (0, 0)),
                   pl.BlockSpec((1, D), lambda i: (0, 0))],
        out_shape=[jax.ShapeDtypeStruct((E, D), jnp.bfloat16),
                   jax.ShapeDtypeStruct((1, D), jnp.float32),
                   jax.ShapeDtypeStruct((1, D), jnp.float32)],
    )(Hs, Hd, y, Wsrc, Wdst, We, bsum.reshape(1, D))


def _gate(z, s, q, gm, bmp, E, bm=2000):
    D = z.shape[1]
    H = D // 2
    Hh = H // 2

    def body(z_ref, s_ref, q_ref, g_ref, b_ref, m0_ref, m1_ref):
        a, c = _bn_coeffs(s_ref, q_ref, g_ref, b_ref, E)
        zn = a * z_ref[...].astype(jnp.float32) + c
        hf = zn[:, :H]
        hs = zn[:, H:]
        m = _sigmoid(hf) * _softplus(hs)
        m0_ref[...] = m[:, :Hh]
        m1_ref[...] = m[:, Hh:]

    cmap = lambda i: (0, 0)
    return pl.pallas_call(
        body,
        grid=(E // bm,),
        in_specs=[pl.BlockSpec((bm, D), lambda i: (i, 0))] +
                 [pl.BlockSpec((1, D), cmap)] * 4,
        out_specs=[pl.BlockSpec((bm, Hh), lambda i: (i, 0)),
                   pl.BlockSpec((bm, Hh), lambda i: (i, 0))],
        out_shape=[jax.ShapeDtypeStruct((E, Hh), jnp.float32),
                   jax.ShapeDtypeStruct((E, Hh), jnp.float32)],
    )(z, s, q, gm.reshape(1, D), bmp.reshape(1, D))


def _residual(h, a0, a1, s0, q0, s1, q1, g, b, rows, bm=2000):
    N, C = h.shape
    Hh = C // 2

    def body(h_ref, a0_ref, a1_ref, s0_ref, q0_ref, s1_ref, q1_ref,
             g_ref, b_ref, o_ref):
        g0 = g_ref[:, :Hh]
        g1 = g_ref[:, Hh:]
        b0 = b_ref[:, :Hh]
        b1 = b_ref[:, Hh:]
        ca0, cc0 = _bn_coeffs(s0_ref, q0_ref, g0, b0, rows)
        ca1, cc1 = _bn_coeffs(s1_ref, q1_ref, g1, b1, rows)
        n0 = ca0 * a0_ref[...] + cc0
        n1 = ca1 * a1_ref[...] + cc1
        aggn = jnp.concatenate([n0, n1], axis=1)
        o_ref[...] = _softplus(h_ref[...] + aggn)

    cmap = lambda i: (0, 0)
    return pl.pallas_call(
        body,
        grid=(N // bm,),
        in_specs=[pl.BlockSpec((bm, C), lambda i: (i, 0)),
                  pl.BlockSpec((bm, Hh), lambda i: (i, 0)),
                  pl.BlockSpec((bm, Hh), lambda i: (i, 0)),
                  pl.BlockSpec((1, Hh), cmap), pl.BlockSpec((1, Hh), cmap),
                  pl.BlockSpec((1, Hh), cmap), pl.BlockSpec((1, Hh), cmap),
                  pl.BlockSpec((1, C), cmap), pl.BlockSpec((1, C), cmap)],
        out_specs=pl.BlockSpec((bm, C), lambda i: (i, 0)),
        out_shape=jax.ShapeDtypeStruct((N, C), jnp.float32),
    )(h, a0, a1, s0, q0, s1, q1, g.reshape(1, C), b.reshape(1, C))


def _sc_gather(A, B, src, dst):
    E = src.shape[0]
    D = A.shape[1]
    NW = 32
    per = E // NW
    CH = 200 if D <= 256 else 40
    NCH = per // CH
    mesh = plsc.VectorSubcoreMesh(core_axis_name="c", subcore_axis_name="s")

    @functools.partial(
        pl.kernel, mesh=mesh,
        out_type=[jax.ShapeDtypeStruct((E, D), jnp.float32),
                  jax.ShapeDtypeStruct((E, D), jnp.float32)],
        scratch_types=[pltpu.VMEM((CH,), jnp.int32),
                       pltpu.VMEM((CH,), jnp.int32),
                       pltpu.VMEM((CH, D), jnp.float32),
                       pltpu.VMEM((CH, D), jnp.float32),
                       pltpu.SemaphoreType.DMA,
                       pltpu.SemaphoreType.DMA])
    def k(a_hbm, b_hbm, s_hbm, d_hbm, oa_hbm, ob_hbm, si, di, ba, bb,
          semA, semB):
        wid = lax.axis_index("s") * 2 + lax.axis_index("c")
        base = wid * per

        def chunk(t, carry):
            off = base + t * CH
            pltpu.sync_copy(s_hbm.at[pl.ds(off, CH)], si)
            pltpu.sync_copy(d_hbm.at[pl.ds(off, CH)], di)
            ca = pltpu.async_copy(a_hbm.at[si], ba, semA)
            cb = pltpu.async_copy(b_hbm.at[di], bb, semB)
            ca.wait()
            cb.wait()
            pltpu.sync_copy(ba, oa_hbm.at[pl.ds(off, CH)])
            pltpu.sync_copy(bb, ob_hbm.at[pl.ds(off, CH)])
            return carry

        lax.fori_loop(0, NCH, chunk, 0)

    return k(A, B, src, dst)


def _sc_scatter(m0, m1, dst, N):
    E, Hh = m0.shape
    NS = 16
    per = E // NS
    CH = 200
    NCH = per // CH
    RB = 200
    NRB = N // RB
    NRB_PER = -(-NRB // NS)
    mesh = plsc.VectorSubcoreMesh(core_axis_name="c", subcore_axis_name="s")

    @functools.partial(
        pl.kernel, mesh=mesh,
        out_type=[jax.ShapeDtypeStruct((N, Hh), jnp.float32),
                  jax.ShapeDtypeStruct((N, Hh), jnp.float32)],
        scratch_types=[pltpu.VMEM((CH,), jnp.int32),
                       pltpu.VMEM((CH, Hh), jnp.float32),
                       pltpu.VMEM_SHARED((N, Hh), jnp.float32)])
    def k(m0_hbm, m1_hbm, d_hbm, o0_hbm, o1_hbm, idxv, mbuf, acc):
        cid = lax.axis_index("c")
        sid = lax.axis_index("s")

        def zrow(e, carry):
            for j in range(Hh // 16):
                mbuf[e, pl.ds(j * 16, 16)] = jnp.zeros((16,), jnp.float32)
            return carry

        lax.fori_loop(0, CH, zrow, 0)

        def zcp(t, carry):
            c = sid + t * NS

            @pl.when(c < NRB)
            def _():
                pltpu.sync_copy(mbuf.at[pl.ds(0, RB)],
                                acc.at[pl.ds(c * RB, RB)])

            return carry

        lax.fori_loop(0, NRB_PER, zcp, 0)
        plsc.subcore_barrier()

        def run_half(m_hbm, o_hbm):
            def chunk(t, carry):
                off = sid * per + t * CH
                pltpu.sync_copy(d_hbm.at[pl.ds(off, CH)], idxv)
                pltpu.sync_copy(m_hbm.at[pl.ds(off, CH)], mbuf)
                pltpu.sync_copy(mbuf, acc.at[idxv], add=True)
                return carry

            lax.fori_loop(0, NCH, chunk, 0)
            plsc.subcore_barrier()

            def wb(t, carry):
                c = sid + t * NS

                @pl.when(c < NRB)
                def _():
                    row = c * RB
                    pltpu.sync_copy(acc.at[pl.ds(row, RB)],
                                    mbuf.at[pl.ds(0, RB)])
                    pltpu.sync_copy(mbuf.at[pl.ds(0, RB)],
                                    o_hbm.at[pl.ds(row, RB)])

                return carry

            lax.fori_loop(0, NRB_PER, wb, 0)

        @pl.when(cid == 0)
        def _():
            run_half(m0_hbm, o0_hbm)

        @pl.when(cid == 1)
        def _():
            run_half(m1_hbm, o1_hbm)

    return k(m0, m1, dst)


def kernel(x, edge_index, r, params):
    N, AIN = x.shape
    E = r.shape[0]
    src = edge_index[0]
    dst = edge_index[1]

    Wa, ba, ga, bba = params["atom"]
    p = _matmul(x, Wa, ba)
    s, q = _stats(p)
    h = _bn_relu(p, s, q, ga, bba, N)

    rb = _rbf(r)
    W1, b1, g1, be1 = params["e1"]
    p1 = _matmul(rb, W1, b1, bm=2000)
    s, q = _stats(p1)
    y = _bn_relu(p1, s, q, g1, be1, E)
    W2, b2, g2, be2 = params["e2"]
    p2 = _matmul(y, W2, b2, bm=2000)
    s, q = _stats(p2)
    y = _bn_relu(p2, s, q, g2, be2, E, out_dtype=jnp.bfloat16)

    for cp in params["convs"]:
        Hs, Hd = _sc_gather(h, h, src, dst)
        bsum = cp["bsrc"] + cp["bdst"] + cp["be"]
        z, s, q = _ye_stats(Hs, Hd, y, cp["Wsrc"], cp["Wdst"], cp["We"], bsum)
        m0, m1 = _gate(z, s, q, cp["gm"], cp["bm"], E)
        a0, a1 = _sc_scatter(m0, m1, dst, N)
        s0, q0 = _stats(a0, bm=2000)
        s1, q1 = _stats(a1, bm=2000)
        h = _residual(h, a0, a1, s0, q0, s1, q1, cp["g2"], cp["b2"], N)

    W1f, b1f = params["fc1"]
    W2f, b2f = params["fc2"]
    W3f, b3f = params["fc3"]
    h = _matmul(h, W1f, b1f, act="relu")
    h = _matmul(h, W2f, b2f, act="relu")
    return _matmul_kacc(h, W3f, b3f, bm=200)


# speedup vs baseline: 2.1816x; 1.1004x over previous
import functools

import jax
import jax.numpy as jnp
from jax import lax
from jax.experimental import pallas as pl
from jax.experimental.pallas import tpu as pltpu
from jax.experimental.pallas import tpu_sc as plsc

_EPS = 1e-5


def _matmul(x, W, b, act=None, bm=1000, bn=None):
    M, K = x.shape
    Nout = W.shape[0]
    if bn is None:
        bn = Nout
    gm, gn = M // bm, Nout // bn
    b2 = b.reshape(1, Nout)

    def body(x_ref, w_ref, b_ref, o_ref):
        acc = lax.dot_general(x_ref[...], w_ref[...], (((1,), (1,)), ((), ())),
                              preferred_element_type=jnp.float32)
        acc = acc + b_ref[...]
        if act == "relu":
            acc = jnp.maximum(acc, 0.0)
        o_ref[...] = acc

    return pl.pallas_call(
        body,
        grid=(gn, gm),
        in_specs=[pl.BlockSpec((bm, K), lambda n, m: (m, 0)),
                  pl.BlockSpec((bn, K), lambda n, m: (n, 0)),
                  pl.BlockSpec((1, bn), lambda n, m: (0, n))],
        out_specs=pl.BlockSpec((bm, bn), lambda n, m: (m, n)),
        out_shape=jax.ShapeDtypeStruct((M, Nout), jnp.float32),
    )(x, W, b2)


def _matmul_kacc(x, W, b, bm=1000, bk=256):
    M, K = x.shape
    Nout = W.shape[0]
    gm = M // bm

    def body(x_ref, w_ref, b_ref, o_ref):
        acc = lax.dot_general(x_ref[...], w_ref[...], (((1,), (1,)), ((), ())),
                              preferred_element_type=jnp.float32)
        acc = jnp.maximum(acc + b_ref[...], 0.0)
        o_ref[...] = acc.reshape(bm, 100, 100)

    return pl.pallas_call(
        body,
        grid=(gm,),
        in_specs=[pl.BlockSpec((bm, K), lambda m: (m, 0)),
                  pl.BlockSpec((Nout, K), lambda m: (0, 0)),
                  pl.BlockSpec((1, Nout), lambda m: (0, 0))],
        out_specs=pl.BlockSpec((bm, 100, 100), lambda m: (m, 0, 0),
                               pipeline_mode=pl.Buffered(buffer_count=1)),
        out_shape=jax.ShapeDtypeStruct((M, 100, 100), jnp.float32),
    )(x.astype(jnp.bfloat16), W.astype(jnp.bfloat16), b.reshape(1, Nout))


def _stats(p, bm=2000):
    R, C = p.shape
    g = R // bm

    def body(p_ref, s_ref, q_ref):
        i = pl.program_id(0)
        t = p_ref[...]
        ts = jnp.sum(t, axis=0, keepdims=True)
        tq = jnp.sum(t * t, axis=0, keepdims=True)

        @pl.when(i == 0)
        def _():
            s_ref[...] = ts
            q_ref[...] = tq

        @pl.when(i > 0)
        def _():
            s_ref[...] += ts
            q_ref[...] += tq

    return pl.pallas_call(
        body,
        grid=(g,),
        in_specs=[pl.BlockSpec((bm, C), lambda i: (i, 0))],
        out_specs=[pl.BlockSpec((1, C), lambda i: (0, 0)),
                   pl.BlockSpec((1, C), lambda i: (0, 0))],
        out_shape=[jax.ShapeDtypeStruct((1, C), jnp.float32),
                   jax.ShapeDtypeStruct((1, C), jnp.float32)],
    )(p)


def _bn_coeffs(s_ref, q_ref, g_ref, b_ref, rows):
    mean = s_ref[...] / rows
    var = q_ref[...] / rows - mean * mean
    a = g_ref[...] / jnp.sqrt(var + _EPS)
    c = b_ref[...] - a * mean
    return a, c


def _pack2(hf):
    C = hf.shape[1]
    Hh = C // 2
    lo = lax.bitcast_convert_type(
        hf[:, :Hh].astype(jnp.bfloat16).astype(jnp.float32), jnp.int32)
    hi = lax.bitcast_convert_type(
        hf[:, Hh:].astype(jnp.bfloat16).astype(jnp.float32), jnp.int32)
    return jnp.bitwise_or(lax.shift_right_logical(lo, 16),
                          jnp.bitwise_and(hi, jnp.int32(-65536)))


def _unpack2(v):
    f_lo = lax.bitcast_convert_type(lax.shift_left(v, 16), jnp.float32)
    f_hi = lax.bitcast_convert_type(jnp.bitwise_and(v, jnp.int32(-65536)),
                                    jnp.float32)
    return jnp.concatenate([f_lo, f_hi], axis=1)


def _softplus(x):
    return jnp.maximum(x, 0.0) + jnp.log1p(jnp.exp(-jnp.abs(x)))


def _sigmoid(x):
    return 1.0 / (1.0 + jnp.exp(-x))


def _bn_relu(p, s, q, g, b, rows, bm=2000, out_dtype=jnp.float32):
    R, C = p.shape

    def body(p_ref, s_ref, q_ref, g_ref, b_ref, o_ref):
        a, c = _bn_coeffs(s_ref, q_ref, g_ref, b_ref, rows)
        o_ref[...] = jnp.maximum(a * p_ref[...] + c, 0.0).astype(out_dtype)

    cmap = lambda i: (0, 0)
    return pl.pallas_call(
        body,
        grid=(R // bm,),
        in_specs=[pl.BlockSpec((bm, C), lambda i: (i, 0))] +
                 [pl.BlockSpec((1, C), cmap)] * 4,
        out_specs=pl.BlockSpec((bm, C), lambda i: (i, 0)),
        out_shape=jax.ShapeDtypeStruct((R, C), out_dtype),
    )(p, s, q, g.reshape(1, C), b.reshape(1, C))


def _rbf(r, EIN=80, bm=2000):
    E = r.shape[0]
    step = 8.0 / (EIN - 1)
    gamma = 1.0 / step

    def body(r_ref, o_ref):
        rt = r_ref[...]
        d = jnp.sqrt(jnp.sum(rt * rt, axis=1, keepdims=True))
        centers = lax.broadcasted_iota(jnp.int32, (1, EIN), 1).astype(jnp.float32) * step
        diff = d - centers
        o_ref[...] = jnp.exp(-gamma * diff * diff)

    return pl.pallas_call(
        body,
        grid=(E // bm,),
        in_specs=[pl.BlockSpec((bm, 3), lambda i: (i, 0))],
        out_specs=pl.BlockSpec((bm, EIN), lambda i: (i, 0)),
        out_shape=jax.ShapeDtypeStruct((E, EIN), jnp.float32),
    )(r)


def _ye_stats(Hs, Hd, y, Wsrc, Wdst, We, bsum, bm=2000):
    E = Hs.shape[0]
    K = y.shape[1]
    D = We.shape[0]
    dn = (((1,), (1,)), ((), ()))

    def body(hs_ref, hd_ref, y_ref, ws_ref, wd_ref, we_ref, b_ref,
             z_ref, s_ref, q_ref):
        i = pl.program_id(0)
        hs = _unpack2(hs_ref[...]).astype(jnp.bfloat16)
        hd = _unpack2(hd_ref[...]).astype(jnp.bfloat16)
        zt = lax.dot_general(hs, ws_ref[...].astype(jnp.bfloat16),
                             dn, preferred_element_type=jnp.float32)
        zt += lax.dot_general(hd, wd_ref[...].astype(jnp.bfloat16),
                              dn, preferred_element_type=jnp.float32)
        zt += lax.dot_general(y_ref[...], we_ref[...].astype(jnp.bfloat16), dn,
                              preferred_element_type=jnp.float32)
        zt += b_ref[...]
        z_ref[...] = zt.astype(jnp.bfloat16)
        ts = jnp.sum(zt, axis=0, keepdims=True)
        tq = jnp.sum(zt * zt, axis=0, keepdims=True)

        @pl.when(i == 0)
        def _():
            s_ref[...] = ts
            q_ref[...] = tq

        @pl.when(i > 0)
        def _():
            s_ref[...] += ts
            q_ref[...] += tq

    return pl.pallas_call(
        body,
        grid=(E // bm,),
        in_specs=[pl.BlockSpec((bm, K // 2), lambda i: (i, 0)),
                  pl.BlockSpec((bm, K // 2), lambda i: (i, 0)),
                  pl.BlockSpec((bm, K), lambda i: (i, 0)),
                  pl.BlockSpec((D, K), lambda i: (0, 0)),
                  pl.BlockSpec((D, K), lambda i: (0, 0)),
                  pl.BlockSpec((D, K), lambda i: (0, 0)),
                  pl.BlockSpec((1, D), lambda i: (0, 0))],
        out_specs=[pl.BlockSpec((bm, D), lambda i: (i, 0)),
                   pl.BlockSpec((1, D), lambda i: (0, 0)),
                   pl.BlockSpec((1, D), lambda i: (0, 0))],
        out_shape=[jax.ShapeDtypeStruct((E, D), jnp.bfloat16),
                   jax.ShapeDtypeStruct((1, D), jnp.float32),
                   jax.ShapeDtypeStruct((1, D), jnp.float32)],
    )(Hs, Hd, y, Wsrc, Wdst, We, bsum.reshape(1, D))


def _gate(z, s, q, gm, bmp, E, bm=2000):
    D = z.shape[1]
    H = D // 2
    Hh = H // 2

    def body(z_ref, s_ref, q_ref, g_ref, b_ref, m0_ref, m1_ref):
        a, c = _bn_coeffs(s_ref, q_ref, g_ref, b_ref, E)
        zn = a * z_ref[...].astype(jnp.float32) + c
        hf = zn[:, :H]
        hs = zn[:, H:]
        m = _sigmoid(hf) * _softplus(hs)
        m0_ref[...] = m[:, :Hh]
        m1_ref[...] = m[:, Hh:]

    cmap = lambda i: (0, 0)
    return pl.pallas_call(
        body,
        grid=(E // bm,),
        in_specs=[pl.BlockSpec((bm, D), lambda i: (i, 0))] +
                 [pl.BlockSpec((1, D), cmap)] * 4,
        out_specs=[pl.BlockSpec((bm, Hh), lambda i: (i, 0)),
                   pl.BlockSpec((bm, Hh), lambda i: (i, 0))],
        out_shape=[jax.ShapeDtypeStruct((E, Hh), jnp.float32),
                   jax.ShapeDtypeStruct((E, Hh), jnp.float32)],
    )(z, s, q, gm.reshape(1, D), bmp.reshape(1, D))


def _residual(h, a0, a1, s0, q0, s1, q1, g, b, rows, bm=2000):
    N, C = h.shape
    Hh = C // 2

    def body(h_ref, a0_ref, a1_ref, s0_ref, q0_ref, s1_ref, q1_ref,
             g_ref, b_ref, o_ref, ob_ref):
        g0 = g_ref[:, :Hh]
        g1 = g_ref[:, Hh:]
        b0 = b_ref[:, :Hh]
        b1 = b_ref[:, Hh:]
        ca0, cc0 = _bn_coeffs(s0_ref, q0_ref, g0, b0, rows)
        ca1, cc1 = _bn_coeffs(s1_ref, q1_ref, g1, b1, rows)
        n0 = ca0 * a0_ref[...] + cc0
        n1 = ca1 * a1_ref[...] + cc1
        aggn = jnp.concatenate([n0, n1], axis=1)
        hn = _softplus(h_ref[...] + aggn)
        o_ref[...] = hn
        ob_ref[...] = _pack2(hn)

    cmap = lambda i: (0, 0)
    return pl.pallas_call(
        body,
        grid=(N // bm,),
        in_specs=[pl.BlockSpec((bm, C), lambda i: (i, 0)),
                  pl.BlockSpec((bm, Hh), lambda i: (i, 0)),
                  pl.BlockSpec((bm, Hh), lambda i: (i, 0)),
                  pl.BlockSpec((1, Hh), cmap), pl.BlockSpec((1, Hh), cmap),
                  pl.BlockSpec((1, Hh), cmap), pl.BlockSpec((1, Hh), cmap),
                  pl.BlockSpec((1, C), cmap), pl.BlockSpec((1, C), cmap)],
        out_specs=[pl.BlockSpec((bm, C), lambda i: (i, 0)),
                   pl.BlockSpec((bm, C // 2), lambda i: (i, 0))],
        out_shape=[jax.ShapeDtypeStruct((N, C), jnp.float32),
                   jax.ShapeDtypeStruct((N, C // 2), jnp.int32)],
    )(h, a0, a1, s0, q0, s1, q1, g.reshape(1, C), b.reshape(1, C))


def _to_b3(h, bm=2000):
    N, C = h.shape

    def body(h_ref, o_ref):
        o_ref[...] = _pack2(h_ref[...])

    return pl.pallas_call(
        body,
        grid=(N // bm,),
        in_specs=[pl.BlockSpec((bm, C), lambda i: (i, 0))],
        out_specs=pl.BlockSpec((bm, C // 2), lambda i: (i, 0)),
        out_shape=jax.ShapeDtypeStruct((N, C // 2), jnp.int32),
    )(h)


def _sc_gather(A, B, src, dst):
    E = src.shape[0]
    T, D = A.shape
    NW = 32
    per = E // NW
    CH = 200
    NCH = per // CH
    mesh = plsc.VectorSubcoreMesh(core_axis_name="c", subcore_axis_name="s")

    @functools.partial(
        pl.kernel, mesh=mesh,
        out_type=[jax.ShapeDtypeStruct((E, D), jnp.int32),
                  jax.ShapeDtypeStruct((E, D), jnp.int32)],
        scratch_types=[pltpu.VMEM((CH,), jnp.int32),
                       pltpu.VMEM((CH,), jnp.int32),
                       pltpu.VMEM((CH, D), jnp.int32),
                       pltpu.VMEM((CH, D), jnp.int32),
                       pltpu.SemaphoreType.DMA,
                       pltpu.SemaphoreType.DMA])
    def k(a_hbm, b_hbm, s_hbm, d_hbm, oa_hbm, ob_hbm, si, di, ba, bb,
          semA, semB):
        wid = lax.axis_index("s") * 2 + lax.axis_index("c")
        base = wid * per

        def chunk(t, carry):
            off = base + t * CH
            pltpu.sync_copy(s_hbm.at[pl.ds(off, CH)], si)
            pltpu.sync_copy(d_hbm.at[pl.ds(off, CH)], di)
            ca = pltpu.async_copy(a_hbm.at[si], ba, semA)
            cb = pltpu.async_copy(b_hbm.at[di], bb, semB)
            ca.wait()
            cb.wait()
            pltpu.sync_copy(ba, oa_hbm.at[pl.ds(off, CH)])
            pltpu.sync_copy(bb, ob_hbm.at[pl.ds(off, CH)])
            return carry

        lax.fori_loop(0, NCH, chunk, 0)

    return k(A, B, src, dst)


def _sc_scatter(m0, m1, dst, N):
    E, Hh = m0.shape
    NS = 16
    per = E // NS
    CH = 200
    NCH = per // CH
    RB = 200
    NRB = N // RB
    NRB_PER = -(-NRB // NS)
    mesh = plsc.VectorSubcoreMesh(core_axis_name="c", subcore_axis_name="s")

    @functools.partial(
        pl.kernel, mesh=mesh,
        out_type=[jax.ShapeDtypeStruct((N, Hh), jnp.float32),
                  jax.ShapeDtypeStruct((N, Hh), jnp.float32)],
        scratch_types=[pltpu.VMEM((CH,), jnp.int32),
                       pltpu.VMEM((CH, Hh), jnp.float32),
                       pltpu.VMEM_SHARED((N, Hh), jnp.float32)])
    def k(m0_hbm, m1_hbm, d_hbm, o0_hbm, o1_hbm, idxv, mbuf, acc):
        cid = lax.axis_index("c")
        sid = lax.axis_index("s")

        def zrow(e, carry):
            for j in range(Hh // 16):
                mbuf[e, pl.ds(j * 16, 16)] = jnp.zeros((16,), jnp.float32)
            return carry

        lax.fori_loop(0, CH, zrow, 0)

        def zcp(t, carry):
            c = sid + t * NS

            @pl.when(c < NRB)
            def _():
                pltpu.sync_copy(mbuf.at[pl.ds(0, RB)],
                                acc.at[pl.ds(c * RB, RB)])

            return carry

        lax.fori_loop(0, NRB_PER, zcp, 0)
        plsc.subcore_barrier()

        def run_half(m_hbm, o_hbm):
            def chunk(t, carry):
                off = sid * per + t * CH
                pltpu.sync_copy(d_hbm.at[pl.ds(off, CH)], idxv)
                pltpu.sync_copy(m_hbm.at[pl.ds(off, CH)], mbuf)
                pltpu.sync_copy(mbuf, acc.at[idxv], add=True)
                return carry

            lax.fori_loop(0, NCH, chunk, 0)
            plsc.subcore_barrier()

            def wb(t, carry):
                c = sid + t * NS

                @pl.when(c < NRB)
                def _():
                    row = c * RB
                    pltpu.sync_copy(acc.at[pl.ds(row, RB)],
                                    mbuf.at[pl.ds(0, RB)])
                    pltpu.sync_copy(mbuf.at[pl.ds(0, RB)],
                                    o_hbm.at[pl.ds(row, RB)])

                return carry

            lax.fori_loop(0, NRB_PER, wb, 0)

        @pl.when(cid == 0)
        def _():
            run_half(m0_hbm, o0_hbm)

        @pl.when(cid == 1)
        def _():
            run_half(m1_hbm, o1_hbm)

    return k(m0, m1, dst)


def kernel(x, edge_index, r, params):
    N, AIN = x.shape
    E = r.shape[0]
    src = edge_index[0]
    dst = edge_index[1]

    Wa, ba, ga, bba = params["atom"]
    p = _matmul(x, Wa, ba)
    s, q = _stats(p)
    h = _bn_relu(p, s, q, ga, bba, N)

    rb = _rbf(r)
    W1, b1, g1, be1 = params["e1"]
    p1 = _matmul(rb, W1, b1, bm=2000)
    s, q = _stats(p1)
    y = _bn_relu(p1, s, q, g1, be1, E)
    W2, b2, g2, be2 = params["e2"]
    p2 = _matmul(y, W2, b2, bm=2000)
    s, q = _stats(p2)
    y = _bn_relu(p2, s, q, g2, be2, E, out_dtype=jnp.bfloat16)

    hb = _to_b3(h)
    for cp in params["convs"]:
        Hs, Hd = _sc_gather(hb, hb, src, dst)
        bsum = cp["bsrc"] + cp["bdst"] + cp["be"]
        z, s, q = _ye_stats(Hs, Hd, y, cp["Wsrc"], cp["Wdst"], cp["We"], bsum)
        m0, m1 = _gate(z, s, q, cp["gm"], cp["bm"], E)
        a0, a1 = _sc_scatter(m0, m1, dst, N)
        s0, q0 = _stats(a0, bm=2000)
        s1, q1 = _stats(a1, bm=2000)
        h, hb = _residual(h, a0, a1, s0, q0, s1, q1, cp["g2"], cp["b2"], N)

    W1f, b1f = params["fc1"]
    W2f, b2f = params["fc2"]
    W3f, b3f = params["fc3"]
    h = _matmul(h, W1f, b1f, act="relu")
    h = _matmul(h, W2f, b2f, act="relu")
    return _matmul_kacc(h, W3f, b3f, bm=200)


# speedup vs baseline: 2.1906x; 1.0041x over previous
import functools

import jax
import jax.numpy as jnp
from jax import lax
from jax.experimental import pallas as pl
from jax.experimental.pallas import tpu as pltpu
from jax.experimental.pallas import tpu_sc as plsc

_EPS = 1e-5


def _matmul(x, W, b, act=None, bm=1000, bn=None):
    M, K = x.shape
    Nout = W.shape[0]
    if bn is None:
        bn = Nout
    gm, gn = M // bm, Nout // bn
    b2 = b.reshape(1, Nout)

    def body(x_ref, w_ref, b_ref, o_ref):
        acc = lax.dot_general(x_ref[...], w_ref[...], (((1,), (1,)), ((), ())),
                              preferred_element_type=jnp.float32)
        acc = acc + b_ref[...]
        if act == "relu":
            acc = jnp.maximum(acc, 0.0)
        o_ref[...] = acc

    return pl.pallas_call(
        body,
        grid=(gn, gm),
        in_specs=[pl.BlockSpec((bm, K), lambda n, m: (m, 0)),
                  pl.BlockSpec((bn, K), lambda n, m: (n, 0)),
                  pl.BlockSpec((1, bn), lambda n, m: (0, n))],
        out_specs=pl.BlockSpec((bm, bn), lambda n, m: (m, n)),
        out_shape=jax.ShapeDtypeStruct((M, Nout), jnp.float32),
    )(x, W, b2)


def _matmul_kacc(x, W, b, bm=1000, bk=256):
    M, K = x.shape
    Nout = W.shape[0]
    gm = M // bm

    def body(x_ref, w_ref, b_ref, o_ref):
        acc = lax.dot_general(x_ref[...], w_ref[...], (((1,), (1,)), ((), ())),
                              preferred_element_type=jnp.float32)
        acc = jnp.maximum(acc + b_ref[...], 0.0)
        o_ref[...] = acc.reshape(bm, 100, 100)

    return pl.pallas_call(
        body,
        grid=(gm,),
        in_specs=[pl.BlockSpec((bm, K), lambda m: (m, 0)),
                  pl.BlockSpec((Nout, K), lambda m: (0, 0)),
                  pl.BlockSpec((1, Nout), lambda m: (0, 0))],
        out_specs=pl.BlockSpec((bm, 100, 100), lambda m: (m, 0, 0),
                               pipeline_mode=pl.Buffered(buffer_count=1)),
        out_shape=jax.ShapeDtypeStruct((M, 100, 100), jnp.float32),
    )(x.astype(jnp.bfloat16), W.astype(jnp.bfloat16), b.reshape(1, Nout))


def _stats(p, bm=2000):
    R, C = p.shape
    g = R // bm

    def body(p_ref, s_ref, q_ref):
        i = pl.program_id(0)
        t = p_ref[...]
        ts = jnp.sum(t, axis=0, keepdims=True)
        tq = jnp.sum(t * t, axis=0, keepdims=True)

        @pl.when(i == 0)
        def _():
            s_ref[...] = ts
            q_ref[...] = tq

        @pl.when(i > 0)
        def _():
            s_ref[...] += ts
            q_ref[...] += tq

    return pl.pallas_call(
        body,
        grid=(g,),
        in_specs=[pl.BlockSpec((bm, C), lambda i: (i, 0))],
        out_specs=[pl.BlockSpec((1, C), lambda i: (0, 0)),
                   pl.BlockSpec((1, C), lambda i: (0, 0))],
        out_shape=[jax.ShapeDtypeStruct((1, C), jnp.float32),
                   jax.ShapeDtypeStruct((1, C), jnp.float32)],
    )(p)


def _bn_coeffs(s_ref, q_ref, g_ref, b_ref, rows, s2=None, q2=None):
    s = s_ref[...] + (s2[...] if s2 is not None else 0.0)
    q = q_ref[...] + (q2[...] if q2 is not None else 0.0)
    mean = s / rows
    var = q / rows - mean * mean
    a = g_ref[...] / jnp.sqrt(var + _EPS)
    c = b_ref[...] - a * mean
    return a, c


def _pack2(hf):
    C = hf.shape[1]
    Hh = C // 2
    lo = lax.bitcast_convert_type(
        hf[:, :Hh].astype(jnp.bfloat16).astype(jnp.float32), jnp.int32)
    hi = lax.bitcast_convert_type(
        hf[:, Hh:].astype(jnp.bfloat16).astype(jnp.float32), jnp.int32)
    return jnp.bitwise_or(lax.shift_right_logical(lo, 16),
                          jnp.bitwise_and(hi, jnp.int32(-65536)))


def _unpack2(v):
    f_lo = lax.bitcast_convert_type(lax.shift_left(v, 16), jnp.float32)
    f_hi = lax.bitcast_convert_type(jnp.bitwise_and(v, jnp.int32(-65536)),
                                    jnp.float32)
    return jnp.concatenate([f_lo, f_hi], axis=1)


def _softplus(x):
    return jnp.maximum(x, 0.0) + jnp.log1p(jnp.exp(-jnp.abs(x)))


def _sigmoid(x):
    return 1.0 / (1.0 + jnp.exp(-x))


def _bn_relu(p, s, q, g, b, rows, bm=2000, out_dtype=jnp.float32):
    R, C = p.shape

    def body(p_ref, s_ref, q_ref, g_ref, b_ref, o_ref):
        a, c = _bn_coeffs(s_ref, q_ref, g_ref, b_ref, rows)
        o_ref[...] = jnp.maximum(a * p_ref[...] + c, 0.0).astype(out_dtype)

    cmap = lambda i: (0, 0)
    return pl.pallas_call(
        body,
        grid=(R // bm,),
        in_specs=[pl.BlockSpec((bm, C), lambda i: (i, 0))] +
                 [pl.BlockSpec((1, C), cmap)] * 4,
        out_specs=pl.BlockSpec((bm, C), lambda i: (i, 0)),
        out_shape=jax.ShapeDtypeStruct((R, C), out_dtype),
    )(p, s, q, g.reshape(1, C), b.reshape(1, C))


def _rbf(r, EIN=80, bm=2000):
    E = r.shape[0]
    step = 8.0 / (EIN - 1)
    gamma = 1.0 / step

    def body(r_ref, o_ref):
        rt = r_ref[...]
        d = jnp.sqrt(jnp.sum(rt * rt, axis=1, keepdims=True))
        centers = lax.broadcasted_iota(jnp.int32, (1, EIN), 1).astype(jnp.float32) * step
        diff = d - centers
        o_ref[...] = jnp.exp(-gamma * diff * diff)

    return pl.pallas_call(
        body,
        grid=(E // bm,),
        in_specs=[pl.BlockSpec((bm, 3), lambda i: (i, 0))],
        out_specs=pl.BlockSpec((bm, EIN), lambda i: (i, 0)),
        out_shape=jax.ShapeDtypeStruct((E, EIN), jnp.float32),
    )(r)


def _ye_stats(Hs, Hd, y, Wsrc, Wdst, We, bsum, bm=1600, yoff=0):
    E = Hs.shape[0]
    K = y.shape[1]
    D = We.shape[0]
    yo = yoff // bm
    dn = (((1,), (1,)), ((), ()))

    def body(hs_ref, hd_ref, y_ref, ws_ref, wd_ref, we_ref, b_ref,
             z_ref, s_ref, q_ref):
        i = pl.program_id(0)
        hs = _unpack2(hs_ref[...]).astype(jnp.bfloat16)
        hd = _unpack2(hd_ref[...]).astype(jnp.bfloat16)
        zt = lax.dot_general(hs, ws_ref[...].astype(jnp.bfloat16),
                             dn, preferred_element_type=jnp.float32)
        zt += lax.dot_general(hd, wd_ref[...].astype(jnp.bfloat16),
                              dn, preferred_element_type=jnp.float32)
        zt += lax.dot_general(y_ref[...], we_ref[...].astype(jnp.bfloat16), dn,
                              preferred_element_type=jnp.float32)
        zt += b_ref[...]
        z_ref[...] = zt.astype(jnp.bfloat16)
        ts = jnp.sum(zt, axis=0, keepdims=True)
        tq = jnp.sum(zt * zt, axis=0, keepdims=True)

        @pl.when(i == 0)
        def _():
            s_ref[...] = ts
            q_ref[...] = tq

        @pl.when(i > 0)
        def _():
            s_ref[...] += ts
            q_ref[...] += tq

    return pl.pallas_call(
        body,
        grid=(E // bm,),
        in_specs=[pl.BlockSpec((bm, K // 2), lambda i: (i, 0)),
                  pl.BlockSpec((bm, K // 2), lambda i: (i, 0)),
                  pl.BlockSpec((bm, K), lambda i: (i + yo, 0)),
                  pl.BlockSpec((D, K), lambda i: (0, 0)),
                  pl.BlockSpec((D, K), lambda i: (0, 0)),
                  pl.BlockSpec((D, K), lambda i: (0, 0)),
                  pl.BlockSpec((1, D), lambda i: (0, 0))],
        out_specs=[pl.BlockSpec((bm, D), lambda i: (i, 0)),
                   pl.BlockSpec((1, D), lambda i: (0, 0)),
                   pl.BlockSpec((1, D), lambda i: (0, 0))],
        out_shape=[jax.ShapeDtypeStruct((E, D), jnp.bfloat16),
                   jax.ShapeDtypeStruct((1, D), jnp.float32),
                   jax.ShapeDtypeStruct((1, D), jnp.float32)],
    )(Hs, Hd, y, Wsrc, Wdst, We, bsum.reshape(1, D))


def _gate(za, zb, sa, qa, sb, qb, gm, bmp, E, bm=1600):
    D = za.shape[1]
    H = D // 2
    Hh = H // 2
    na = za.shape[0] // bm

    def body(za_ref, zb_ref, sa_ref, qa_ref, sb_ref, qb_ref,
             g_ref, b_ref, m0_ref, m1_ref):
        i = pl.program_id(0)
        a, c = _bn_coeffs(sa_ref, qa_ref, g_ref, b_ref, E, s2=sb_ref,
                          q2=qb_ref)
        zt = jnp.where(i < na, za_ref[...].astype(jnp.float32),
                       zb_ref[...].astype(jnp.float32))
        zn = a * zt + c
        hf = zn[:, :H]
        hs = zn[:, H:]
        m = _sigmoid(hf) * _softplus(hs)
        m0_ref[...] = m[:, :Hh]
        m1_ref[...] = m[:, Hh:]

    cmap = lambda i: (0, 0)
    return pl.pallas_call(
        body,
        grid=(E // bm,),
        in_specs=[pl.BlockSpec((bm, D), lambda i: (jnp.minimum(i, na - 1), 0)),
                  pl.BlockSpec((bm, D), lambda i: (jnp.maximum(i - na, 0), 0))] +
                 [pl.BlockSpec((1, D), cmap)] * 6,
        out_specs=[pl.BlockSpec((bm, Hh), lambda i: (i, 0)),
                   pl.BlockSpec((bm, Hh), lambda i: (i, 0))],
        out_shape=[jax.ShapeDtypeStruct((E, Hh), jnp.float32),
                   jax.ShapeDtypeStruct((E, Hh), jnp.float32)],
    )(za, zb, sa, qa, sb, qb, gm.reshape(1, D), bmp.reshape(1, D))


def _residual(h, a0, a1, s0, q0, s1, q1, g, b, rows, bm=2000):
    N, C = h.shape
    Hh = C // 2

    def body(h_ref, a0_ref, a1_ref, s0_ref, q0_ref, s1_ref, q1_ref,
             g_ref, b_ref, o_ref, ob_ref):
        g0 = g_ref[:, :Hh]
        g1 = g_ref[:, Hh:]
        b0 = b_ref[:, :Hh]
        b1 = b_ref[:, Hh:]
        ca0, cc0 = _bn_coeffs(s0_ref, q0_ref, g0, b0, rows)
        ca1, cc1 = _bn_coeffs(s1_ref, q1_ref, g1, b1, rows)
        n0 = ca0 * a0_ref[...] + cc0
        n1 = ca1 * a1_ref[...] + cc1
        aggn = jnp.concatenate([n0, n1], axis=1)
        hn = _softplus(h_ref[...] + aggn)
        o_ref[...] = hn
        ob_ref[...] = _pack2(hn)

    cmap = lambda i: (0, 0)
    return pl.pallas_call(
        body,
        grid=(N // bm,),
        in_specs=[pl.BlockSpec((bm, C), lambda i: (i, 0)),
                  pl.BlockSpec((bm, Hh), lambda i: (i, 0)),
                  pl.BlockSpec((bm, Hh), lambda i: (i, 0)),
                  pl.BlockSpec((1, Hh), cmap), pl.BlockSpec((1, Hh), cmap),
                  pl.BlockSpec((1, Hh), cmap), pl.BlockSpec((1, Hh), cmap),
                  pl.BlockSpec((1, C), cmap), pl.BlockSpec((1, C), cmap)],
        out_specs=[pl.BlockSpec((bm, C), lambda i: (i, 0)),
                   pl.BlockSpec((bm, C // 2), lambda i: (i, 0))],
        out_shape=[jax.ShapeDtypeStruct((N, C), jnp.float32),
                   jax.ShapeDtypeStruct((N, C // 2), jnp.int32)],
    )(h, a0, a1, s0, q0, s1, q1, g.reshape(1, C), b.reshape(1, C))


def _to_b3(h, bm=2000):
    N, C = h.shape

    def body(h_ref, o_ref):
        o_ref[...] = _pack2(h_ref[...])

    return pl.pallas_call(
        body,
        grid=(N // bm,),
        in_specs=[pl.BlockSpec((bm, C), lambda i: (i, 0))],
        out_specs=pl.BlockSpec((bm, C // 2), lambda i: (i, 0)),
        out_shape=jax.ShapeDtypeStruct((N, C // 2), jnp.int32),
    )(h)


def _sc_gather(A, B, src, dst):
    E = src.shape[0]
    T, D = A.shape
    NW = 32
    per = E // NW
    CH = 200
    NCH = per // CH
    mesh = plsc.VectorSubcoreMesh(core_axis_name="c", subcore_axis_name="s")

    @functools.partial(
        pl.kernel, mesh=mesh,
        out_type=[jax.ShapeDtypeStruct((E, D), jnp.int32),
                  jax.ShapeDtypeStruct((E, D), jnp.int32)],
        scratch_types=[pltpu.VMEM((CH,), jnp.int32),
                       pltpu.VMEM((CH,), jnp.int32),
                       pltpu.VMEM((CH, D), jnp.int32),
                       pltpu.VMEM((CH, D), jnp.int32),
                       pltpu.SemaphoreType.DMA,
                       pltpu.SemaphoreType.DMA])
    def k(a_hbm, b_hbm, s_hbm, d_hbm, oa_hbm, ob_hbm, si, di, ba, bb,
          semA, semB):
        wid = lax.axis_index("s") * 2 + lax.axis_index("c")
        base = wid * per

        def chunk(t, carry):
            off = base + t * CH
            pltpu.sync_copy(s_hbm.at[pl.ds(off, CH)], si)
            pltpu.sync_copy(d_hbm.at[pl.ds(off, CH)], di)
            ca = pltpu.async_copy(a_hbm.at[si], ba, semA)
            cb = pltpu.async_copy(b_hbm.at[di], bb, semB)
            ca.wait()
            cb.wait()
            pltpu.sync_copy(ba, oa_hbm.at[pl.ds(off, CH)])
            pltpu.sync_copy(bb, ob_hbm.at[pl.ds(off, CH)])
            return carry

        lax.fori_loop(0, NCH, chunk, 0)

    return k(A, B, src, dst)


def _sc_scatter(m0, m1, dst, N):
    E, Hh = m0.shape
    NS = 16
    per = E // NS
    CH = 200
    NCH = per // CH
    RB = 200
    NRB = N // RB
    NRB_PER = -(-NRB // NS)
    mesh = plsc.VectorSubcoreMesh(core_axis_name="c", subcore_axis_name="s")

    @functools.partial(
        pl.kernel, mesh=mesh,
        out_type=[jax.ShapeDtypeStruct((N, Hh), jnp.float32),
                  jax.ShapeDtypeStruct((N, Hh), jnp.float32)],
        scratch_types=[pltpu.VMEM((CH,), jnp.int32),
                       pltpu.VMEM((CH, Hh), jnp.float32),
                       pltpu.VMEM_SHARED((N, Hh), jnp.float32)])
    def k(m0_hbm, m1_hbm, d_hbm, o0_hbm, o1_hbm, idxv, mbuf, acc):
        cid = lax.axis_index("c")
        sid = lax.axis_index("s")

        def zrow(e, carry):
            for j in range(Hh // 16):
                mbuf[e, pl.ds(j * 16, 16)] = jnp.zeros((16,), jnp.float32)
            return carry

        lax.fori_loop(0, CH, zrow, 0)

        def zcp(t, carry):
            c = sid + t * NS

            @pl.when(c < NRB)
            def _():
                pltpu.sync_copy(mbuf.at[pl.ds(0, RB)],
                                acc.at[pl.ds(c * RB, RB)])

            return carry

        lax.fori_loop(0, NRB_PER, zcp, 0)
        plsc.subcore_barrier()

        def run_half(m_hbm, o_hbm):
            def chunk(t, carry):
                off = sid * per + t * CH
                pltpu.sync_copy(d_hbm.at[pl.ds(off, CH)], idxv)
                pltpu.sync_copy(m_hbm.at[pl.ds(off, CH)], mbuf)
                pltpu.sync_copy(mbuf, acc.at[idxv], add=True)
                return carry

            lax.fori_loop(0, NCH, chunk, 0)
            plsc.subcore_barrier()

            def wb(t, carry):
                c = sid + t * NS

                @pl.when(c < NRB)
                def _():
                    row = c * RB
                    pltpu.sync_copy(acc.at[pl.ds(row, RB)],
                                    mbuf.at[pl.ds(0, RB)])
                    pltpu.sync_copy(mbuf.at[pl.ds(0, RB)],
                                    o_hbm.at[pl.ds(row, RB)])

                return carry

            lax.fori_loop(0, NRB_PER, wb, 0)

        @pl.when(cid == 0)
        def _():
            run_half(m0_hbm, o0_hbm)

        @pl.when(cid == 1)
        def _():
            run_half(m1_hbm, o1_hbm)

    return k(m0, m1, dst)


def kernel(x, edge_index, r, params):
    N, AIN = x.shape
    E = r.shape[0]
    src = edge_index[0]
    dst = edge_index[1]

    Wa, ba, ga, bba = params["atom"]
    p = _matmul(x, Wa, ba)
    s, q = _stats(p)
    h = _bn_relu(p, s, q, ga, bba, N)

    rb = _rbf(r)
    W1, b1, g1, be1 = params["e1"]
    p1 = _matmul(rb, W1, b1, bm=2000)
    s, q = _stats(p1)
    y = _bn_relu(p1, s, q, g1, be1, E)
    W2, b2, g2, be2 = params["e2"]
    p2 = _matmul(y, W2, b2, bm=2000)
    s, q = _stats(p2)
    y = _bn_relu(p2, s, q, g2, be2, E, out_dtype=jnp.bfloat16)

    hb = _to_b3(h)
    E1 = 76800
    src_a, src_b = src[:E1], src[E1:]
    dst_a, dst_b = dst[:E1], dst[E1:]
    for cp in params["convs"]:
        Hs_a, Hd_a = _sc_gather(hb, hb, src_a, dst_a)
        Hs_b, Hd_b = _sc_gather(hb, hb, src_b, dst_b)
        bsum = cp["bsrc"] + cp["bdst"] + cp["be"]
        za, sa, qa = _ye_stats(Hs_a, Hd_a, y, cp["Wsrc"], cp["Wdst"],
                               cp["We"], bsum)
        zb, sb, qb = _ye_stats(Hs_b, Hd_b, y, cp["Wsrc"], cp["Wdst"],
                               cp["We"], bsum, yoff=E1)
        m0, m1 = _gate(za, zb, sa, qa, sb, qb, cp["gm"], cp["bm"], E)
        a0, a1 = _sc_scatter(m0, m1, dst, N)
        s0, q0 = _stats(a0, bm=2000)
        s1, q1 = _stats(a1, bm=2000)
        h, hb = _residual(h, a0, a1, s0, q0, s1, q1, cp["g2"], cp["b2"], N)

    W1f, b1f = params["fc1"]
    W2f, b2f = params["fc2"]
    W3f, b3f = params["fc3"]
    h = _matmul(h, W1f, b1f, act="relu")
    h = _matmul(h, W2f, b2f, act="relu")
    return _matmul_kacc(h, W3f, b3f, bm=200)
